# baseline (speedup 1.0000x reference)
import functools

import jax
import jax.numpy as jnp
from jax import lax
from jax.experimental import pallas as pl
from jax.experimental.pallas import tpu as pltpu
from jax.experimental.pallas import tpu_sc as plsc

N_MEM = 10000
E = 160000
B = 64
N_ENT = 100000
D = 128
N_REL = 500

NW = 32
CH = 128
EC = 40
EW = CH * EC
EPAD = NW * EW
NSEG = 10240
NC_NODE = 3
NWN = CH * NC_NODE
NPAD = NW * NWN


def _tc_pre_body(hid, wm1a, wm1c, relp, wm1b, qh, qr, wm1d, wm1e, bm1,
                 wh1d, wh1e, bh1, wint, wh1c,
                 h1_o, h2_o, rt_o, qm_o, qhn_o, wic_o):
    h = hid[...]
    h1_o[...] = jnp.dot(h, wm1a[...], preferred_element_type=jnp.float32)
    h2_o[...] = jnp.dot(h, wm1c[...], preferred_element_type=jnp.float32)

    @pl.when(pl.program_id(0) == 0)
    def _():
        rt_o[...] = jnp.dot(relp[...], wm1b[...], preferred_element_type=jnp.float32)
        qhv = qh[...]
        qrv = qr[...]
        qm_o[...] = (jnp.dot(qhv, wm1d[...], preferred_element_type=jnp.float32)
                     + jnp.dot(qrv, wm1e[...], preferred_element_type=jnp.float32)
                     + bm1[...])
        qhn_o[...] = (jnp.dot(qhv, wh1d[...], preferred_element_type=jnp.float32)
                      + jnp.dot(qrv, wh1e[...], preferred_element_type=jnp.float32)
                      + bh1[...])
        wic_o[...] = jnp.dot(wint[...], wh1c[...], preferred_element_type=jnp.float32)


ROWS_W = 640


def _sc_tab_gather_body(tab_hbm, idx_hbm, out_hbm,
                        idx_v, ba, bb, sh_tab, sa, sb):
    cid = lax.axis_index("c")
    sid = lax.axis_index("s")
    wid = sid * 2 + cid

    pltpu.sync_copy(idx_hbm.at[wid], idx_v)
    pltpu.sync_copy(tab_hbm.at[pl.ds(sid * ROWS_W, ROWS_W)],
                    sh_tab.at[pl.ds(sid * ROWS_W, ROWS_W)])
    plsc.subcore_barrier()

    gb = (ba, bb)
    sg = (sa, sb)

    def start(c, ph):
        pltpu.async_copy(sh_tab.at[idx_v.at[c]], gb[ph], sg[ph])

    def finish(c, ph):
        pltpu.make_async_copy(sh_tab.at[idx_v.at[c]], gb[ph], sg[ph]).wait()
        pltpu.sync_copy(gb[ph], out_hbm.at[pl.ds(wid * EW + c * CH, CH)])

    start(0, 0)

    def pair_body(p, carry):
        c = p * 2
        start(c + 1, 1)
        finish(c, 0)
        start(c + 2, 0)
        finish(c + 1, 1)
        return carry

    lax.fori_loop(0, EC // 2 - 1, pair_body, 0)
    c = EC - 2
    start(c + 1, 1)
    finish(c, 0)
    finish(c + 1, 1)


def _sc_node_body(v_hbm, a_hbm, hu_hbm, att_hbm, hu_out, attg_out,
                  vds, ads, hb0, hb1, hb2, ab0, ab1, ab2, sh_sem, sa_sem):
    cid = lax.axis_index("c")
    sid = lax.axis_index("s")
    wid = sid * 2 + cid

    pltpu.sync_copy(v_hbm.at[wid], vds)
    pltpu.sync_copy(a_hbm.at[wid], ads)
    hb = (hb0, hb1, hb2)
    ab = (ab0, ab1, ab2)
    for c in range(NC_NODE):
        pltpu.async_copy(hu_hbm.at[vds.at[c]], hb[c], sh_sem)
        pltpu.async_copy(att_hbm.at[ads.at[c]], ab[c], sa_sem)
    for c in range(NC_NODE):
        pltpu.make_async_copy(hu_hbm.at[vds.at[c]], hb[c], sh_sem).wait()
        pltpu.sync_copy(hb[c], hu_out.at[pl.ds(wid * NWN + c * CH, CH)])
        pltpu.make_async_copy(att_hbm.at[ads.at[c]], ab[c], sa_sem).wait()
        pltpu.sync_copy(ab[c], attg_out.at[pl.ds(wid * NWN + c * CH, CH)])


def _tc_msg_body(pre_ref, pre2_ref, rel_ref, eg_ref, rt, qm, w2, b2, out_ref):
    ohr = (rel_ref[...] == lax.broadcasted_iota(jnp.int32, (1, 512), 1)
           ).astype(jnp.float32)
    ohe = (eg_ref[...] == lax.broadcasted_iota(jnp.int32, (1, B), 1)
           ).astype(jnp.float32)
    x = (pre_ref[...] + pre2_ref[...]
         + jnp.dot(ohr, rt[...], preferred_element_type=jnp.float32)
         + jnp.dot(ohe, qm[...], preferred_element_type=jnp.float32))
    y = jnp.where(x >= 0, x, 0.01 * x)
    z = jnp.dot(y, w2[...], preferred_element_type=jnp.float32) + b2[...]
    out_ref[...] = jnp.tanh(z)


def _sc_scatter_body(msg_hbm, js_hbm, seg_out, cnt_out,
                     msgbuf, msgbuf2, jds, ones1, zc,
                     sh_seg, sh_cnt, sm0, sm1):
    cid = lax.axis_index("c")
    sid = lax.axis_index("s")
    wid = sid * 2 + cid
    zero16 = jnp.zeros((16,), jnp.float32)
    one16 = jnp.ones((16,), jnp.float32)

    def zrow(r, rc):
        for k in range(8):
            msgbuf[r, pl.ds(k * 16, 16)] = zero16
        return rc

    lax.fori_loop(0, CH, zrow, 0)

    def fill1(r, rc):
        ones1[pl.ds(r * 16, 16)] = one16
        zc[pl.ds(r * 16, 16)] = zero16
        zc[pl.ds((r + 8) * 16, 16)] = zero16
        zc[pl.ds((r + 16) * 16, 16)] = zero16
        zc[pl.ds((r + 24) * 16, 16)] = zero16
        zc[pl.ds((r + 32) * 16, 16)] = zero16
        return rc

    lax.fori_loop(0, 8, fill1, 0)

    def zseg(t, rc):
        pltpu.sync_copy(msgbuf, sh_seg.at[pl.ds(sid * 640 + t * CH, CH)])
        return rc

    lax.fori_loop(0, 5, zseg, 0)
    pltpu.sync_copy(zc, sh_cnt.at[pl.ds(sid * 640, 640)])

    plsc.subcore_barrier()

    pltpu.sync_copy(js_hbm.at[wid], jds)

    mb = (msgbuf, msgbuf2)
    sm = (sm0, sm1)

    def start(c, ph):
        pltpu.async_copy(msg_hbm.at[pl.ds(wid * EW + c * CH, CH)], mb[ph], sm[ph])

    def finish(c, ph):
        pltpu.make_async_copy(msg_hbm.at[pl.ds(wid * EW + c * CH, CH)],
                              mb[ph], sm[ph]).wait()
        pltpu.sync_copy(mb[ph], sh_seg.at[jds.at[c]], add=True)
        pltpu.sync_copy(ones1, sh_cnt.at[jds.at[c]], add=True)

    start(0, 0)

    def pair_body(p, rc):
        c = p * 2
        start(c + 1, 1)
        finish(c, 0)
        start(c + 2, 0)
        finish(c + 1, 1)
        return rc

    lax.fori_loop(0, EC // 2 - 1, pair_body, 0)
    c2 = EC - 2
    start(c2 + 1, 1)
    finish(c2, 0)
    finish(c2 + 1, 1)

    plsc.subcore_barrier()

    pltpu.sync_copy(sh_seg.at[pl.ds(sid * 640, 640)],
                    seg_out.at[cid, pl.ds(sid * 640, 640)])
    pltpu.sync_copy(sh_cnt.at[pl.ds(sid * 640, 640)],
                    cnt_out.at[cid, pl.ds(sid * 640, 640)])


def _tc_final_body(seg_ref, cnt0_ref, cnt1_ref, hid_ref, hu_ref, att_ref,
                   meg_ref, wh1a, wh1b, wic, qh1, wh2, bh2, out_ref):
    seg = seg_ref[0] + seg_ref[1]
    cnt = cnt0_ref[...] + cnt1_ref[...]
    ma = seg * lax.rsqrt(jnp.maximum(cnt, 1.0))
    hid = hid_ref[...]
    hu = att_ref[...] * hu_ref[...]
    oh = (meg_ref[...] == lax.broadcasted_iota(jnp.int32, (1, B), 1)
          ).astype(jnp.float32)
    pre2 = (jnp.dot(ma, wh1a[...], preferred_element_type=jnp.float32)
            + jnp.dot(hid, wh1b[...], preferred_element_type=jnp.float32)
            + jnp.dot(hu, wic[...], preferred_element_type=jnp.float32)
            + jnp.dot(oh, qh1[...], preferred_element_type=jnp.float32))
    y = jnp.where(pre2 >= 0, pre2, 0.01 * pre2)
    out_ref[...] = hid + jnp.tanh(
        jnp.dot(y, wh2[...], preferred_element_type=jnp.float32) + bh2[...])


def _full(shape):
    return pl.BlockSpec(shape, lambda i: (0,) * len(shape))


def kernel(hidden, seen_edges, memorized_nodes, node_attention, hidden_uncon,
           query_head_emb, query_rel_emb, rel_table,
           W_m1, b_m1, W_m2, b_m2, W_h1, b_h1, W_h2, b_h2, W_int):
    f32 = jnp.float32

    eg = seen_edges[:, 0]
    rel = seen_edges[:, 3]
    jcol = seen_edges[:, 5]
    vi_row = seen_edges[:, 6]

    def pad_idx(x, value):
        return jnp.pad(x, (0, EPAD - E), constant_values=value).reshape(NW, EC, CH)

    vi_g = pad_idx(vi_row, 0)
    j_g = pad_idx(jcol, 0)
    j_s = pad_idx(jcol, NSEG - 1)

    mem_eg = memorized_nodes[:, 0]
    v = memorized_nodes[:, 1]
    v_p = jnp.pad(v, (0, NPAD - N_MEM)).reshape(NW, NC_NODE, CH)
    aflat = mem_eg * N_ENT + v
    a_p = jnp.pad(aflat, (0, NPAD - N_MEM)).reshape(NW, NC_NODE, CH)
    att_flat = node_attention.reshape(B * N_ENT)
    hu_tab = hidden_uncon.reshape(N_ENT, D)

    rel_pad = jnp.pad(rel_table, ((0, 512 - N_REL), (0, 0)))
    bm1 = b_m1.reshape(1, D)
    bm2 = b_m2.reshape(1, D)
    bh1 = b_h1.reshape(1, D)
    bh2 = b_h2.reshape(1, D)
    meg2 = jnp.pad(mem_eg, (0, NSEG - N_MEM)).reshape(NSEG, 1)

    hid_pad = jnp.pad(hidden, ((0, NSEG - N_MEM), (0, 0)))
    BL1 = 1024
    h1, h2, rt, qm1, qh1, wic = pl.pallas_call(
        _tc_pre_body,
        grid=(NSEG // BL1,),
        in_specs=[
            pl.BlockSpec((BL1, D), lambda i: (i, 0)),
            _full((D, D)), _full((D, D)), _full((512, D)), _full((D, D)),
            _full((B, D)), _full((B, D)), _full((D, D)), _full((D, D)),
            _full((1, D)), _full((D, D)), _full((D, D)), _full((1, D)),
            _full((D, D)), _full((D, D)),
        ],
        out_specs=[
            pl.BlockSpec((BL1, D), lambda i: (i, 0)),
            pl.BlockSpec((BL1, D), lambda i: (i, 0)),
            _full((512, D)), _full((B, D)), _full((B, D)), _full((D, D)),
        ],
        out_shape=[
            jax.ShapeDtypeStruct((NSEG, D), f32),
            jax.ShapeDtypeStruct((NSEG, D), f32),
            jax.ShapeDtypeStruct((512, D), f32),
            jax.ShapeDtypeStruct((B, D), f32),
            jax.ShapeDtypeStruct((B, D), f32),
            jax.ShapeDtypeStruct((D, D), f32),
        ],
    )(hid_pad, W_m1[0:D], W_m1[2 * D:3 * D], rel_pad, W_m1[D:2 * D],
      query_head_emb, query_rel_emb, W_m1[3 * D:4 * D], W_m1[4 * D:5 * D], bm1,
      W_h1[3 * D:4 * D], W_h1[4 * D:5 * D], bh1, W_int, W_h1[2 * D:3 * D])

    mesh = plsc.VectorSubcoreMesh(core_axis_name="c", subcore_axis_name="s")

    def tab_gather(tab, idx):
        return pl.kernel(
            _sc_tab_gather_body,
            out_type=jax.ShapeDtypeStruct((EPAD, D), f32),
            mesh=mesh,
            scratch_types=[
                pltpu.VMEM((EC, CH), jnp.int32),
                pltpu.VMEM((CH, D), f32),
                pltpu.VMEM((CH, D), f32),
                pltpu.VMEM_SHARED((NSEG, D), f32),
                pltpu.SemaphoreType.DMA,
                pltpu.SemaphoreType.DMA,
            ],
        )(tab, idx)

    pre = tab_gather(h1, vi_g)
    pre2 = tab_gather(h2, j_g)

    hu_g, att_g = pl.kernel(
        _sc_node_body,
        out_type=[
            jax.ShapeDtypeStruct((NPAD, D), f32),
            jax.ShapeDtypeStruct((NPAD,), f32),
        ],
        mesh=mesh,
        scratch_types=[
            pltpu.VMEM((NC_NODE, CH), jnp.int32),
            pltpu.VMEM((NC_NODE, CH), jnp.int32),
            pltpu.VMEM((CH, D), f32),
            pltpu.VMEM((CH, D), f32),
            pltpu.VMEM((CH, D), f32),
            pltpu.VMEM((CH,), f32),
            pltpu.VMEM((CH,), f32),
            pltpu.VMEM((CH,), f32),
            pltpu.SemaphoreType.DMA,
            pltpu.SemaphoreType.DMA,
        ],
    )(v_p, a_p, hu_tab, att_flat)

    BL3 = 512
    rel_col = jnp.pad(rel, (0, EPAD - E)).reshape(EPAD, 1)
    eg_col = jnp.pad(eg, (0, EPAD - E)).reshape(EPAD, 1)
    msg = pl.pallas_call(
        _tc_msg_body,
        grid=(EPAD // BL3,),
        in_specs=[
            pl.BlockSpec((BL3, D), lambda i: (i, 0)),
            pl.BlockSpec((BL3, D), lambda i: (i, 0)),
            pl.BlockSpec((BL3, 1), lambda i: (i, 0)),
            pl.BlockSpec((BL3, 1), lambda i: (i, 0)),
            _full((512, D)), _full((B, D)),
            _full((D, D)), _full((1, D)),
        ],
        out_specs=pl.BlockSpec((BL3, D), lambda i: (i, 0)),
        out_shape=jax.ShapeDtypeStruct((EPAD, D), f32),
    )(pre, pre2, rel_col, eg_col, rt, qm1, W_m2, bm2)

    seg, cntp = pl.kernel(
        _sc_scatter_body,
        out_type=[
            jax.ShapeDtypeStruct((2, NSEG, D), f32),
            jax.ShapeDtypeStruct((2, NSEG), f32),
        ],
        mesh=mesh,
        scratch_types=[
            pltpu.VMEM((CH, D), f32),
            pltpu.VMEM((CH, D), f32),
            pltpu.VMEM((EC, CH), jnp.int32),
            pltpu.VMEM((CH,), f32),
            pltpu.VMEM((640,), f32),
            pltpu.VMEM_SHARED((NSEG, D), f32),
            pltpu.VMEM_SHARED((NSEG,), f32),
            pltpu.SemaphoreType.DMA,
            pltpu.SemaphoreType.DMA,
        ],
    )(msg, j_s)

    BL5 = 1024
    nblk = NSEG // BL5
    cnt_flat = cntp.reshape(2 * NSEG, 1)
    out = pl.pallas_call(
        _tc_final_body,
        grid=(nblk,),
        in_specs=[
            pl.BlockSpec((2, BL5, D), lambda i: (0, i, 0)),
            pl.BlockSpec((BL5, 1), lambda i: (i, 0)),
            pl.BlockSpec((BL5, 1), lambda i: (i + nblk, 0)),
            pl.BlockSpec((BL5, D), lambda i: (i, 0)),
            pl.BlockSpec((BL5, D), lambda i: (i, 0)),
            pl.BlockSpec((BL5, 1), lambda i: (i, 0)),
            pl.BlockSpec((BL5, 1), lambda i: (i, 0)),
            _full((D, D)), _full((D, D)), _full((D, D)),
            _full((B, D)), _full((D, D)), _full((1, D)),
        ],
        out_specs=pl.BlockSpec((BL5, D), lambda i: (i, 0)),
        out_shape=jax.ShapeDtypeStruct((NSEG, D), f32),
    )(seg, cnt_flat, cnt_flat, hid_pad, hu_g, att_g.reshape(-1, 1), meg2,
      W_h1[0:D], W_h1[D:2 * D], wic, qh1, W_h2, bh2)

    return out[:N_MEM]


# speedup vs baseline: 1.0021x; 1.0021x over previous
import functools

import jax
import jax.numpy as jnp
from jax import lax
from jax.experimental import pallas as pl
from jax.experimental.pallas import tpu as pltpu
from jax.experimental.pallas import tpu_sc as plsc

N_MEM = 10000
E = 160000
B = 64
N_ENT = 100000
D = 128
N_REL = 500

NW = 32
CH = 128
EC = 40
EW = CH * EC
EPAD = NW * EW
NSEG = 10240
NC_NODE = 3
NWN = CH * NC_NODE
NPAD = NW * NWN


def _tc_pre_body(hid, wm1a, wm1c, relp, wm1b, qh, qr, wm1d, wm1e, bm1,
                 wh1d, wh1e, bh1, wint, wh1c,
                 h1_o, h2_o, rt_o, qm_o, qhn_o, wic_o):
    h = hid[...]
    h1_o[...] = jnp.dot(h, wm1a[...], preferred_element_type=jnp.float32)
    h2_o[...] = jnp.dot(h, wm1c[...], preferred_element_type=jnp.float32)

    @pl.when(pl.program_id(0) == 0)
    def _():
        rt_o[...] = jnp.dot(relp[...], wm1b[...], preferred_element_type=jnp.float32)
        qhv = qh[...]
        qrv = qr[...]
        qm_o[...] = (jnp.dot(qhv, wm1d[...], preferred_element_type=jnp.float32)
                     + jnp.dot(qrv, wm1e[...], preferred_element_type=jnp.float32)
                     + bm1[...])
        qhn_o[...] = (jnp.dot(qhv, wh1d[...], preferred_element_type=jnp.float32)
                      + jnp.dot(qrv, wh1e[...], preferred_element_type=jnp.float32)
                      + bh1[...])
        wic_o[...] = jnp.dot(wint[...], wh1c[...], preferred_element_type=jnp.float32)


ROWS_W = 640


def _sc_tab_gather_body(tab_hbm, idx_hbm, out_hbm,
                        idx_v, ba, bb, sh_tab, sa, sb):
    cid = lax.axis_index("c")
    sid = lax.axis_index("s")
    wid = sid * 2 + cid

    pltpu.sync_copy(idx_hbm.at[wid], idx_v)
    pltpu.sync_copy(tab_hbm.at[pl.ds(sid * ROWS_W, ROWS_W)],
                    sh_tab.at[pl.ds(sid * ROWS_W, ROWS_W)])
    plsc.subcore_barrier()

    gb = (ba, bb)
    sg = (sa, sb)

    def start(c, ph):
        pltpu.async_copy(sh_tab.at[idx_v.at[c]], gb[ph], sg[ph])

    def finish(c, ph):
        pltpu.make_async_copy(sh_tab.at[idx_v.at[c]], gb[ph], sg[ph]).wait()
        pltpu.sync_copy(gb[ph], out_hbm.at[pl.ds(wid * EW + c * CH, CH)])

    start(0, 0)

    def pair_body(p, carry):
        c = p * 2
        start(c + 1, 1)
        finish(c, 0)
        start(c + 2, 0)
        finish(c + 1, 1)
        return carry

    lax.fori_loop(0, EC // 2 - 1, pair_body, 0)
    c = EC - 2
    start(c + 1, 1)
    finish(c, 0)
    finish(c + 1, 1)


def _sc_node_body(v_hbm, a_hbm, hu_hbm, att_hbm, hu_out, attg_out,
                  vds, ads, hb0, hb1, hb2, ab0, ab1, ab2, sh_sem, sa_sem):
    cid = lax.axis_index("c")
    sid = lax.axis_index("s")
    wid = sid * 2 + cid

    pltpu.sync_copy(v_hbm.at[wid], vds)
    pltpu.sync_copy(a_hbm.at[wid], ads)
    hb = (hb0, hb1, hb2)
    ab = (ab0, ab1, ab2)
    for c in range(NC_NODE):
        pltpu.async_copy(hu_hbm.at[vds.at[c]], hb[c], sh_sem)
        pltpu.async_copy(att_hbm.at[ads.at[c]], ab[c], sa_sem)
    for c in range(NC_NODE):
        pltpu.make_async_copy(hu_hbm.at[vds.at[c]], hb[c], sh_sem).wait()
        pltpu.sync_copy(hb[c], hu_out.at[pl.ds(wid * NWN + c * CH, CH)])
        pltpu.make_async_copy(att_hbm.at[ads.at[c]], ab[c], sa_sem).wait()
        pltpu.sync_copy(ab[c], attg_out.at[pl.ds(wid * NWN + c * CH, CH)])


def _tc_msg_body(pre_ref, pre2_ref, rel_ref, eg_ref, rt, qm, w2, b2, out_ref):
    bf = jnp.bfloat16
    ohr = (rel_ref[...] == lax.broadcasted_iota(jnp.int32, (1, 512), 1)
           ).astype(bf)
    ohe = (eg_ref[...] == lax.broadcasted_iota(jnp.int32, (1, B), 1)
           ).astype(bf)
    x = (pre_ref[...] + pre2_ref[...]
         + jnp.dot(ohr, rt[...].astype(bf), preferred_element_type=jnp.float32)
         + jnp.dot(ohe, qm[...].astype(bf), preferred_element_type=jnp.float32))
    y = jnp.where(x >= 0, x, 0.01 * x)
    z = jnp.dot(y.astype(bf), w2[...].astype(bf),
                preferred_element_type=jnp.float32) + b2[...]
    out_ref[...] = jnp.tanh(z)


def _sc_scatter_body(msg_hbm, js_hbm, seg_out, cnt_out,
                     msgbuf, msgbuf2, jds, ones1, zc,
                     sh_seg, sh_cnt, sm0, sm1):
    cid = lax.axis_index("c")
    sid = lax.axis_index("s")
    wid = sid * 2 + cid
    zero16 = jnp.zeros((16,), jnp.float32)
    one16 = jnp.ones((16,), jnp.float32)

    def zrow(r, rc):
        for k in range(8):
            msgbuf[r, pl.ds(k * 16, 16)] = zero16
        return rc

    lax.fori_loop(0, CH, zrow, 0)

    def fill1(r, rc):
        ones1[pl.ds(r * 16, 16)] = one16
        zc[pl.ds(r * 16, 16)] = zero16
        zc[pl.ds((r + 8) * 16, 16)] = zero16
        zc[pl.ds((r + 16) * 16, 16)] = zero16
        zc[pl.ds((r + 24) * 16, 16)] = zero16
        zc[pl.ds((r + 32) * 16, 16)] = zero16
        return rc

    lax.fori_loop(0, 8, fill1, 0)

    def zseg(t, rc):
        pltpu.sync_copy(msgbuf, sh_seg.at[pl.ds(sid * 640 + t * CH, CH)])
        return rc

    lax.fori_loop(0, 5, zseg, 0)
    pltpu.sync_copy(zc, sh_cnt.at[pl.ds(sid * 640, 640)])

    plsc.subcore_barrier()

    pltpu.sync_copy(js_hbm.at[wid], jds)

    mb = (msgbuf, msgbuf2)
    sm = (sm0, sm1)

    def start(c, ph):
        pltpu.async_copy(msg_hbm.at[pl.ds(wid * EW + c * CH, CH)], mb[ph], sm[ph])

    def finish(c, ph):
        pltpu.make_async_copy(msg_hbm.at[pl.ds(wid * EW + c * CH, CH)],
                              mb[ph], sm[ph]).wait()
        pltpu.sync_copy(mb[ph], sh_seg.at[jds.at[c]], add=True)
        pltpu.sync_copy(ones1, sh_cnt.at[jds.at[c]], add=True)

    start(0, 0)

    def pair_body(p, rc):
        c = p * 2
        start(c + 1, 1)
        finish(c, 0)
        start(c + 2, 0)
        finish(c + 1, 1)
        return rc

    lax.fori_loop(0, EC // 2 - 1, pair_body, 0)
    c2 = EC - 2
    start(c2 + 1, 1)
    finish(c2, 0)
    finish(c2 + 1, 1)

    plsc.subcore_barrier()

    pltpu.sync_copy(sh_seg.at[pl.ds(sid * 640, 640)],
                    seg_out.at[cid, pl.ds(sid * 640, 640)])
    pltpu.sync_copy(sh_cnt.at[pl.ds(sid * 640, 640)],
                    cnt_out.at[cid, pl.ds(sid * 640, 640)])


def _tc_final_body(seg_ref, cnt0_ref, cnt1_ref, hid_ref, hu_ref, att_ref,
                   meg_ref, wh1a, wh1b, wic, qh1, wh2, bh2, out_ref):
    seg = seg_ref[0] + seg_ref[1]
    cnt = cnt0_ref[...] + cnt1_ref[...]
    ma = seg * lax.rsqrt(jnp.maximum(cnt, 1.0))
    hid = hid_ref[...]
    hu = att_ref[...] * hu_ref[...]
    oh = (meg_ref[...] == lax.broadcasted_iota(jnp.int32, (1, B), 1)
          ).astype(jnp.float32)
    pre2 = (jnp.dot(ma, wh1a[...], preferred_element_type=jnp.float32)
            + jnp.dot(hid, wh1b[...], preferred_element_type=jnp.float32)
            + jnp.dot(hu, wic[...], preferred_element_type=jnp.float32)
            + jnp.dot(oh, qh1[...], preferred_element_type=jnp.float32))
    y = jnp.where(pre2 >= 0, pre2, 0.01 * pre2)
    out_ref[...] = hid + jnp.tanh(
        jnp.dot(y, wh2[...], preferred_element_type=jnp.float32) + bh2[...])


def _full(shape):
    return pl.BlockSpec(shape, lambda i: (0,) * len(shape))


def kernel(hidden, seen_edges, memorized_nodes, node_attention, hidden_uncon,
           query_head_emb, query_rel_emb, rel_table,
           W_m1, b_m1, W_m2, b_m2, W_h1, b_h1, W_h2, b_h2, W_int):
    f32 = jnp.float32

    eg = seen_edges[:, 0]
    rel = seen_edges[:, 3]
    jcol = seen_edges[:, 5]
    vi_row = seen_edges[:, 6]

    def pad_idx(x, value):
        return jnp.pad(x, (0, EPAD - E), constant_values=value).reshape(NW, EC, CH)

    vi_g = pad_idx(vi_row, 0)
    j_g = pad_idx(jcol, 0)
    j_s = pad_idx(jcol, NSEG - 1)

    mem_eg = memorized_nodes[:, 0]
    v = memorized_nodes[:, 1]
    v_p = jnp.pad(v, (0, NPAD - N_MEM)).reshape(NW, NC_NODE, CH)
    aflat = mem_eg * N_ENT + v
    a_p = jnp.pad(aflat, (0, NPAD - N_MEM)).reshape(NW, NC_NODE, CH)
    att_flat = node_attention.reshape(B * N_ENT)
    hu_tab = hidden_uncon.reshape(N_ENT, D)

    rel_pad = jnp.pad(rel_table, ((0, 512 - N_REL), (0, 0)))
    bm1 = b_m1.reshape(1, D)
    bm2 = b_m2.reshape(1, D)
    bh1 = b_h1.reshape(1, D)
    bh2 = b_h2.reshape(1, D)
    meg2 = jnp.pad(mem_eg, (0, NSEG - N_MEM)).reshape(NSEG, 1)

    hid_pad = jnp.pad(hidden, ((0, NSEG - N_MEM), (0, 0)))
    BL1 = 1024
    h1, h2, rt, qm1, qh1, wic = pl.pallas_call(
        _tc_pre_body,
        grid=(NSEG // BL1,),
        in_specs=[
            pl.BlockSpec((BL1, D), lambda i: (i, 0)),
            _full((D, D)), _full((D, D)), _full((512, D)), _full((D, D)),
            _full((B, D)), _full((B, D)), _full((D, D)), _full((D, D)),
            _full((1, D)), _full((D, D)), _full((D, D)), _full((1, D)),
            _full((D, D)), _full((D, D)),
        ],
        out_specs=[
            pl.BlockSpec((BL1, D), lambda i: (i, 0)),
            pl.BlockSpec((BL1, D), lambda i: (i, 0)),
            _full((512, D)), _full((B, D)), _full((B, D)), _full((D, D)),
        ],
        out_shape=[
            jax.ShapeDtypeStruct((NSEG, D), f32),
            jax.ShapeDtypeStruct((NSEG, D), f32),
            jax.ShapeDtypeStruct((512, D), f32),
            jax.ShapeDtypeStruct((B, D), f32),
            jax.ShapeDtypeStruct((B, D), f32),
            jax.ShapeDtypeStruct((D, D), f32),
        ],
    )(hid_pad, W_m1[0:D], W_m1[2 * D:3 * D], rel_pad, W_m1[D:2 * D],
      query_head_emb, query_rel_emb, W_m1[3 * D:4 * D], W_m1[4 * D:5 * D], bm1,
      W_h1[3 * D:4 * D], W_h1[4 * D:5 * D], bh1, W_int, W_h1[2 * D:3 * D])

    mesh = plsc.VectorSubcoreMesh(core_axis_name="c", subcore_axis_name="s")

    def tab_gather(tab, idx):
        return pl.kernel(
            _sc_tab_gather_body,
            out_type=jax.ShapeDtypeStruct((EPAD, D), f32),
            mesh=mesh,
            scratch_types=[
                pltpu.VMEM((EC, CH), jnp.int32),
                pltpu.VMEM((CH, D), f32),
                pltpu.VMEM((CH, D), f32),
                pltpu.VMEM_SHARED((NSEG, D), f32),
                pltpu.SemaphoreType.DMA,
                pltpu.SemaphoreType.DMA,
            ],
        )(tab, idx)

    pre = tab_gather(h1, vi_g)
    pre2 = tab_gather(h2, j_g)

    hu_g, att_g = pl.kernel(
        _sc_node_body,
        out_type=[
            jax.ShapeDtypeStruct((NPAD, D), f32),
            jax.ShapeDtypeStruct((NPAD,), f32),
        ],
        mesh=mesh,
        scratch_types=[
            pltpu.VMEM((NC_NODE, CH), jnp.int32),
            pltpu.VMEM((NC_NODE, CH), jnp.int32),
            pltpu.VMEM((CH, D), f32),
            pltpu.VMEM((CH, D), f32),
            pltpu.VMEM((CH, D), f32),
            pltpu.VMEM((CH,), f32),
            pltpu.VMEM((CH,), f32),
            pltpu.VMEM((CH,), f32),
            pltpu.SemaphoreType.DMA,
            pltpu.SemaphoreType.DMA,
        ],
    )(v_p, a_p, hu_tab, att_flat)

    BL3 = 512
    rel_col = jnp.pad(rel, (0, EPAD - E)).reshape(EPAD, 1)
    eg_col = jnp.pad(eg, (0, EPAD - E)).reshape(EPAD, 1)
    msg = pl.pallas_call(
        _tc_msg_body,
        grid=(EPAD // BL3,),
        in_specs=[
            pl.BlockSpec((BL3, D), lambda i: (i, 0)),
            pl.BlockSpec((BL3, D), lambda i: (i, 0)),
            pl.BlockSpec((BL3, 1), lambda i: (i, 0)),
            pl.BlockSpec((BL3, 1), lambda i: (i, 0)),
            _full((512, D)), _full((B, D)),
            _full((D, D)), _full((1, D)),
        ],
        out_specs=pl.BlockSpec((BL3, D), lambda i: (i, 0)),
        out_shape=jax.ShapeDtypeStruct((EPAD, D), f32),
    )(pre, pre2, rel_col, eg_col, rt, qm1, W_m2, bm2)

    seg, cntp = pl.kernel(
        _sc_scatter_body,
        out_type=[
            jax.ShapeDtypeStruct((2, NSEG, D), f32),
            jax.ShapeDtypeStruct((2, NSEG), f32),
        ],
        mesh=mesh,
        scratch_types=[
            pltpu.VMEM((CH, D), f32),
            pltpu.VMEM((CH, D), f32),
            pltpu.VMEM((EC, CH), jnp.int32),
            pltpu.VMEM((CH,), f32),
            pltpu.VMEM((640,), f32),
            pltpu.VMEM_SHARED((NSEG, D), f32),
            pltpu.VMEM_SHARED((NSEG,), f32),
            pltpu.SemaphoreType.DMA,
            pltpu.SemaphoreType.DMA,
        ],
    )(msg, j_s)

    BL5 = 1024
    nblk = NSEG // BL5
    cnt_flat = cntp.reshape(2 * NSEG, 1)
    out = pl.pallas_call(
        _tc_final_body,
        grid=(nblk,),
        in_specs=[
            pl.BlockSpec((2, BL5, D), lambda i: (0, i, 0)),
            pl.BlockSpec((BL5, 1), lambda i: (i, 0)),
            pl.BlockSpec((BL5, 1), lambda i: (i + nblk, 0)),
            pl.BlockSpec((BL5, D), lambda i: (i, 0)),
            pl.BlockSpec((BL5, D), lambda i: (i, 0)),
            pl.BlockSpec((BL5, 1), lambda i: (i, 0)),
            pl.BlockSpec((BL5, 1), lambda i: (i, 0)),
            _full((D, D)), _full((D, D)), _full((D, D)),
            _full((B, D)), _full((D, D)), _full((1, D)),
        ],
        out_specs=pl.BlockSpec((BL5, D), lambda i: (i, 0)),
        out_shape=jax.ShapeDtypeStruct((NSEG, D), f32),
    )(seg, cnt_flat, cnt_flat, hid_pad, hu_g, att_g.reshape(-1, 1), meg2,
      W_h1[0:D], W_h1[D:2 * D], wic, qh1, W_h2, bh2)

    return out[:N_MEM]


# speedup vs baseline: 1.1499x; 1.1475x over previous
import functools

import jax
import jax.numpy as jnp
from jax import lax
from jax.experimental import pallas as pl
from jax.experimental.pallas import tpu as pltpu
from jax.experimental.pallas import tpu_sc as plsc

N_MEM = 10000
E = 160000
B = 64
N_ENT = 100000
D = 128
N_REL = 500

NW = 32
CH = 128
EC = 40
EW = CH * EC
EPAD = NW * EW
NSEG = 10240
NC_NODE = 3
NWN = CH * NC_NODE
NPAD = NW * NWN


def _tc_pre_body(hid, wm1a, wm1c, relp, wm1b, qh, qr, wm1d, wm1e, bm1,
                 wh1d, wh1e, bh1, wint, wh1c,
                 h1_o, h2_o, rt_o, qm_o, qhn_o, wic_o):
    h = hid[...]
    h1_o[...] = jnp.dot(h, wm1a[...], preferred_element_type=jnp.float32)
    h2_o[...] = jnp.dot(h, wm1c[...], preferred_element_type=jnp.float32)

    @pl.when(pl.program_id(0) == 0)
    def _():
        rt_o[...] = jnp.dot(relp[...], wm1b[...], preferred_element_type=jnp.float32)
        qhv = qh[...]
        qrv = qr[...]
        qm_o[...] = (jnp.dot(qhv, wm1d[...], preferred_element_type=jnp.float32)
                     + jnp.dot(qrv, wm1e[...], preferred_element_type=jnp.float32)
                     + bm1[...])
        qhn_o[...] = (jnp.dot(qhv, wh1d[...], preferred_element_type=jnp.float32)
                      + jnp.dot(qrv, wh1e[...], preferred_element_type=jnp.float32)
                      + bh1[...])
        wic_o[...] = jnp.dot(wint[...], wh1c[...], preferred_element_type=jnp.float32)


ROWS_W = 640


def _sc_tab_gather_body(tab_hbm, idx_hbm, out_hbm,
                        idx_v, ba, bb, sh_tab, sa, sb):
    cid = lax.axis_index("c")
    sid = lax.axis_index("s")
    wid = sid * 2 + cid

    pltpu.sync_copy(idx_hbm.at[wid], idx_v)
    pltpu.sync_copy(tab_hbm.at[pl.ds(sid * ROWS_W, ROWS_W)],
                    sh_tab.at[pl.ds(sid * ROWS_W, ROWS_W)])
    plsc.subcore_barrier()

    gb = (ba, bb)
    sg = (sa, sb)

    def start(c, ph):
        pltpu.async_copy(sh_tab.at[idx_v.at[c]], gb[ph], sg[ph])

    def finish(c, ph):
        pltpu.make_async_copy(sh_tab.at[idx_v.at[c]], gb[ph], sg[ph]).wait()
        pltpu.sync_copy(gb[ph], out_hbm.at[pl.ds(wid * EW + c * CH, CH)])

    start(0, 0)

    def pair_body(p, carry):
        c = p * 2
        start(c + 1, 1)
        finish(c, 0)
        start(c + 2, 0)
        finish(c + 1, 1)
        return carry

    lax.fori_loop(0, EC // 2 - 1, pair_body, 0)
    c = EC - 2
    start(c + 1, 1)
    finish(c, 0)
    finish(c + 1, 1)


def _sc_node_body(v_hbm, a_hbm, hu_hbm, att_hbm, hu_out, attg_out,
                  vds, ads, hb0, hb1, hb2, ab0, ab1, ab2, sh_sem, sa_sem):
    cid = lax.axis_index("c")
    sid = lax.axis_index("s")
    wid = sid * 2 + cid

    pltpu.sync_copy(v_hbm.at[wid], vds)
    pltpu.sync_copy(a_hbm.at[wid], ads)
    hb = (hb0, hb1, hb2)
    ab = (ab0, ab1, ab2)
    for c in range(NC_NODE):
        pltpu.async_copy(hu_hbm.at[vds.at[c]], hb[c], sh_sem)
        pltpu.async_copy(att_hbm.at[ads.at[c]], ab[c], sa_sem)
    for c in range(NC_NODE):
        pltpu.make_async_copy(hu_hbm.at[vds.at[c]], hb[c], sh_sem).wait()
        pltpu.sync_copy(hb[c], hu_out.at[pl.ds(wid * NWN + c * CH, CH)])
        pltpu.make_async_copy(att_hbm.at[ads.at[c]], ab[c], sa_sem).wait()
        pltpu.sync_copy(ab[c], attg_out.at[pl.ds(wid * NWN + c * CH, CH)])


def _tc_msg_body(pre_ref, pre2_ref, rel_ref, eg_ref, rt, qm, w2, b2, out_ref):
    bf = jnp.bfloat16
    ohr = (rel_ref[...] == lax.broadcasted_iota(jnp.int32, (1, 512), 1)
           ).astype(bf)
    ohe = (eg_ref[...] == lax.broadcasted_iota(jnp.int32, (1, B), 1)
           ).astype(bf)
    x = (pre_ref[...] + pre2_ref[...]
         + jnp.dot(ohr, rt[...].astype(bf), preferred_element_type=jnp.float32)
         + jnp.dot(ohe, qm[...].astype(bf), preferred_element_type=jnp.float32))
    y = jnp.where(x >= 0, x, 0.01 * x)
    z = jnp.dot(y.astype(bf), w2[...].astype(bf),
                preferred_element_type=jnp.float32) + b2[...]
    out_ref[...] = jnp.tanh(z)


def _sc_scatter_body(msg_hbm, js_hbm, seg_out, cnt_out,
                     msgbuf, msgbuf2, jds, ones1, zc,
                     sh_seg, sh_cnt, sm0, sm1):
    cid = lax.axis_index("c")
    sid = lax.axis_index("s")
    wid = sid * 2 + cid
    zero16 = jnp.zeros((16,), jnp.float32)
    one16 = jnp.ones((16,), jnp.float32)

    def zrow(r, rc):
        for k in range(8):
            msgbuf[r, pl.ds(k * 16, 16)] = zero16
        return rc

    lax.fori_loop(0, CH, zrow, 0)

    def fill1(r, rc):
        ones1[pl.ds(r * 16, 16)] = one16
        zc[pl.ds(r * 16, 16)] = zero16
        zc[pl.ds((r + 8) * 16, 16)] = zero16
        zc[pl.ds((r + 16) * 16, 16)] = zero16
        zc[pl.ds((r + 24) * 16, 16)] = zero16
        zc[pl.ds((r + 32) * 16, 16)] = zero16
        return rc

    lax.fori_loop(0, 8, fill1, 0)

    def zseg(t, rc):
        pltpu.sync_copy(msgbuf, sh_seg.at[pl.ds(sid * 640 + t * CH, CH)])
        return rc

    lax.fori_loop(0, 5, zseg, 0)
    pltpu.sync_copy(zc, sh_cnt.at[pl.ds(sid * 640, 640)])

    plsc.subcore_barrier()

    pltpu.sync_copy(js_hbm.at[wid], jds)

    mb = (msgbuf, msgbuf2)
    sm = (sm0, sm1)

    def start(c, ph):
        pltpu.async_copy(msg_hbm.at[pl.ds(wid * EW + c * CH, CH)], mb[ph], sm[ph])

    def finish(c, ph):
        pltpu.make_async_copy(msg_hbm.at[pl.ds(wid * EW + c * CH, CH)],
                              mb[ph], sm[ph]).wait()
        pltpu.sync_copy(mb[ph], sh_seg.at[jds.at[c]], add=True)
        pltpu.sync_copy(ones1, sh_cnt.at[jds.at[c]], add=True)

    start(0, 0)

    def pair_body(p, rc):
        c = p * 2
        start(c + 1, 1)
        finish(c, 0)
        start(c + 2, 0)
        finish(c + 1, 1)
        return rc

    lax.fori_loop(0, EC // 2 - 1, pair_body, 0)
    c2 = EC - 2
    start(c2 + 1, 1)
    finish(c2, 0)
    finish(c2 + 1, 1)

    plsc.subcore_barrier()

    pltpu.sync_copy(sh_seg.at[pl.ds(sid * 640, 640)],
                    seg_out.at[cid, pl.ds(sid * 640, 640)])
    pltpu.sync_copy(sh_cnt.at[pl.ds(sid * 640, 640)],
                    cnt_out.at[cid, pl.ds(sid * 640, 640)])


def _tc_final_body(seg_ref, cnt0_ref, cnt1_ref, hid_ref, hu_ref, att_ref,
                   meg_ref, wh1a, wh1b, wic, qh1, wh2, bh2, out_ref):
    seg = seg_ref[0] + seg_ref[1]
    cnt = cnt0_ref[...] + cnt1_ref[...]
    ma = seg * lax.rsqrt(jnp.maximum(cnt, 1.0))
    hid = hid_ref[...]
    hu = att_ref[...] * hu_ref[...]
    oh = (meg_ref[...] == lax.broadcasted_iota(jnp.int32, (1, B), 1)
          ).astype(jnp.float32)
    pre2 = (jnp.dot(ma, wh1a[...], preferred_element_type=jnp.float32)
            + jnp.dot(hid, wh1b[...], preferred_element_type=jnp.float32)
            + jnp.dot(hu, wic[...], preferred_element_type=jnp.float32)
            + jnp.dot(oh, qh1[...], preferred_element_type=jnp.float32))
    y = jnp.where(pre2 >= 0, pre2, 0.01 * pre2)
    out_ref[...] = hid + jnp.tanh(
        jnp.dot(y, wh2[...], preferred_element_type=jnp.float32) + bh2[...])


def _full(shape):
    return pl.BlockSpec(shape, lambda i: (0,) * len(shape))


def kernel(hidden, seen_edges, memorized_nodes, node_attention, hidden_uncon,
           query_head_emb, query_rel_emb, rel_table,
           W_m1, b_m1, W_m2, b_m2, W_h1, b_h1, W_h2, b_h2, W_int):
    f32 = jnp.float32

    eg = seen_edges[:, 0]
    rel = seen_edges[:, 3]
    jcol = seen_edges[:, 5]
    vi_row = seen_edges[:, 6]

    def pad_idx(x, value):
        return jnp.pad(x, (0, EPAD - E), constant_values=value).reshape(NW, EC, CH)

    vi_g = pad_idx(vi_row, 0)
    j_g = pad_idx(jcol, 0)
    j_s = pad_idx(jcol, NSEG - 1)

    mem_eg = memorized_nodes[:, 0]
    v = memorized_nodes[:, 1]
    v_p = jnp.pad(v, (0, NPAD - N_MEM)).reshape(NW, NC_NODE, CH)
    aflat = mem_eg * N_ENT + v
    a_p = jnp.pad(aflat, (0, NPAD - N_MEM)).reshape(NW, NC_NODE, CH)
    att_flat = node_attention.reshape(B * N_ENT)
    hu_tab = hidden_uncon.reshape(N_ENT, D)

    rel_pad = jnp.pad(rel_table, ((0, 512 - N_REL), (0, 0)))
    bm1 = b_m1.reshape(1, D)
    bm2 = b_m2.reshape(1, D)
    bh1 = b_h1.reshape(1, D)
    bh2 = b_h2.reshape(1, D)
    meg2 = jnp.pad(mem_eg, (0, NSEG - N_MEM)).reshape(NSEG, 1)

    hid_pad = jnp.pad(hidden, ((0, NSEG - N_MEM), (0, 0)))
    BL1 = 1024
    h1, h2, rt, qm1, qh1, wic = pl.pallas_call(
        _tc_pre_body,
        grid=(NSEG // BL1,),
        in_specs=[
            pl.BlockSpec((BL1, D), lambda i: (i, 0)),
            _full((D, D)), _full((D, D)), _full((512, D)), _full((D, D)),
            _full((B, D)), _full((B, D)), _full((D, D)), _full((D, D)),
            _full((1, D)), _full((D, D)), _full((D, D)), _full((1, D)),
            _full((D, D)), _full((D, D)),
        ],
        out_specs=[
            pl.BlockSpec((BL1, D), lambda i: (i, 0)),
            pl.BlockSpec((BL1, D), lambda i: (i, 0)),
            _full((512, D)), _full((B, D)), _full((B, D)), _full((D, D)),
        ],
        out_shape=[
            jax.ShapeDtypeStruct((NSEG, D), f32),
            jax.ShapeDtypeStruct((NSEG, D), f32),
            jax.ShapeDtypeStruct((512, D), f32),
            jax.ShapeDtypeStruct((B, D), f32),
            jax.ShapeDtypeStruct((B, D), f32),
            jax.ShapeDtypeStruct((D, D), f32),
        ],
    )(hid_pad, W_m1[0:D], W_m1[2 * D:3 * D], rel_pad, W_m1[D:2 * D],
      query_head_emb, query_rel_emb, W_m1[3 * D:4 * D], W_m1[4 * D:5 * D], bm1,
      W_h1[3 * D:4 * D], W_h1[4 * D:5 * D], bh1, W_int, W_h1[2 * D:3 * D])

    mesh = plsc.VectorSubcoreMesh(core_axis_name="c", subcore_axis_name="s")

    def tab_gather(tab, idx):
        return pl.kernel(
            _sc_tab_gather_body,
            out_type=jax.ShapeDtypeStruct((EPAD, D), f32),
            mesh=mesh,
            scratch_types=[
                pltpu.VMEM((EC, CH), jnp.int32),
                pltpu.VMEM((CH, D), f32),
                pltpu.VMEM((CH, D), f32),
                pltpu.VMEM_SHARED((NSEG, D), f32),
                pltpu.SemaphoreType.DMA,
                pltpu.SemaphoreType.DMA,
            ],
        )(tab, idx)

    pre = tab_gather(h1, vi_g)
    pre2 = tab_gather(h2, j_g)

    hu_g, att_g = pl.kernel(
        _sc_node_body,
        out_type=[
            jax.ShapeDtypeStruct((NPAD, D), f32),
            jax.ShapeDtypeStruct((NPAD,), f32),
        ],
        mesh=mesh,
        scratch_types=[
            pltpu.VMEM((NC_NODE, CH), jnp.int32),
            pltpu.VMEM((NC_NODE, CH), jnp.int32),
            pltpu.VMEM((CH, D), f32),
            pltpu.VMEM((CH, D), f32),
            pltpu.VMEM((CH, D), f32),
            pltpu.VMEM((CH,), f32),
            pltpu.VMEM((CH,), f32),
            pltpu.VMEM((CH,), f32),
            pltpu.SemaphoreType.DMA,
            pltpu.SemaphoreType.DMA,
        ],
    )(v_p, a_p, hu_tab, att_flat)

    BL3 = 1024
    rel_col = jnp.pad(rel, (0, EPAD - E)).reshape(EPAD, 1)
    eg_col = jnp.pad(eg, (0, EPAD - E)).reshape(EPAD, 1)
    msg = pl.pallas_call(
        _tc_msg_body,
        grid=(EPAD // BL3,),
        in_specs=[
            pl.BlockSpec((BL3, D), lambda i: (i, 0)),
            pl.BlockSpec((BL3, D), lambda i: (i, 0)),
            pl.BlockSpec((BL3, 1), lambda i: (i, 0)),
            pl.BlockSpec((BL3, 1), lambda i: (i, 0)),
            _full((512, D)), _full((B, D)),
            _full((D, D)), _full((1, D)),
        ],
        out_specs=pl.BlockSpec((BL3, D), lambda i: (i, 0)),
        out_shape=jax.ShapeDtypeStruct((EPAD, D), f32),
    )(pre, pre2, rel_col, eg_col, rt, qm1, W_m2, bm2)

    seg, cntp = pl.kernel(
        _sc_scatter_body,
        out_type=[
            jax.ShapeDtypeStruct((2, NSEG, D), f32),
            jax.ShapeDtypeStruct((2, NSEG), f32),
        ],
        mesh=mesh,
        scratch_types=[
            pltpu.VMEM((CH, D), f32),
            pltpu.VMEM((CH, D), f32),
            pltpu.VMEM((EC, CH), jnp.int32),
            pltpu.VMEM((CH,), f32),
            pltpu.VMEM((640,), f32),
            pltpu.VMEM_SHARED((NSEG, D), f32),
            pltpu.VMEM_SHARED((NSEG,), f32),
            pltpu.SemaphoreType.DMA,
            pltpu.SemaphoreType.DMA,
        ],
    )(msg, j_s)

    BL5 = 1024
    nblk = NSEG // BL5
    cnt_flat = cntp.reshape(2 * NSEG, 1)
    out = pl.pallas_call(
        _tc_final_body,
        grid=(nblk,),
        in_specs=[
            pl.BlockSpec((2, BL5, D), lambda i: (0, i, 0)),
            pl.BlockSpec((BL5, 1), lambda i: (i, 0)),
            pl.BlockSpec((BL5, 1), lambda i: (i + nblk, 0)),
            pl.BlockSpec((BL5, D), lambda i: (i, 0)),
            pl.BlockSpec((BL5, D), lambda i: (i, 0)),
            pl.BlockSpec((BL5, 1), lambda i: (i, 0)),
            pl.BlockSpec((BL5, 1), lambda i: (i, 0)),
            _full((D, D)), _full((D, D)), _full((D, D)),
            _full((B, D)), _full((D, D)), _full((1, D)),
        ],
        out_specs=pl.BlockSpec((BL5, D), lambda i: (i, 0)),
        out_shape=jax.ShapeDtypeStruct((NSEG, D), f32),
    )(seg, cnt_flat, cnt_flat, hid_pad, hu_g, att_g.reshape(-1, 1), meg2,
      W_h1[0:D], W_h1[D:2 * D], wic, qh1, W_h2, bh2)

    return out[:N_MEM]


# speedup vs baseline: 1.2463x; 1.0838x over previous
import functools

import jax
import jax.numpy as jnp
from jax import lax
from jax.experimental import pallas as pl
from jax.experimental.pallas import tpu as pltpu
from jax.experimental.pallas import tpu_sc as plsc

N_MEM = 10000
E = 160000
B = 64
N_ENT = 100000
D = 128
N_REL = 500

NW = 32
CH = 128
EC = 40
EW = CH * EC
EPAD = NW * EW
NSEG = 10240
NC_NODE = 3
NWN = CH * NC_NODE
NPAD = NW * NWN


def _tc_pre_body(hid, wm1a, wm1c, relp, wm1b, qh, qr, wm1d, wm1e, bm1,
                 wh1d, wh1e, bh1, wint, wh1c,
                 h1_o, h2_o, rt_o, qm_o, qhn_o, wic_o):
    h = hid[...]
    h1_o[...] = jnp.dot(h, wm1a[...], preferred_element_type=jnp.float32)
    h2_o[...] = jnp.dot(h, wm1c[...], preferred_element_type=jnp.float32)

    @pl.when(pl.program_id(0) == 0)
    def _():
        rt_o[...] = jnp.dot(relp[...], wm1b[...], preferred_element_type=jnp.float32)
        qhv = qh[...]
        qrv = qr[...]
        qm_o[...] = (jnp.dot(qhv, wm1d[...], preferred_element_type=jnp.float32)
                     + jnp.dot(qrv, wm1e[...], preferred_element_type=jnp.float32)
                     + bm1[...])
        qhn_o[...] = (jnp.dot(qhv, wh1d[...], preferred_element_type=jnp.float32)
                      + jnp.dot(qrv, wh1e[...], preferred_element_type=jnp.float32)
                      + bh1[...])
        wic_o[...] = jnp.dot(wint[...], wh1c[...], preferred_element_type=jnp.float32)


ROWS_W = 640


def _sc_tab_gather_body(tab_hbm, idx_hbm, out_hbm,
                        idx_v, ba, bb, sh_tab, sa, sb):
    cid = lax.axis_index("c")
    sid = lax.axis_index("s")
    wid = sid * 2 + cid

    pltpu.sync_copy(idx_hbm.at[wid], idx_v)
    pltpu.sync_copy(tab_hbm.at[pl.ds(sid * ROWS_W, ROWS_W)],
                    sh_tab.at[pl.ds(sid * ROWS_W, ROWS_W)])
    plsc.subcore_barrier()

    gb = (ba, bb)
    sg = (sa, sb)

    def start(c, ph):
        pltpu.async_copy(sh_tab.at[idx_v.at[c]], gb[ph], sg[ph])

    def finish(c, ph):
        pltpu.make_async_copy(sh_tab.at[idx_v.at[c]], gb[ph], sg[ph]).wait()
        pltpu.sync_copy(gb[ph], out_hbm.at[pl.ds(wid * EW + c * CH, CH)])

    start(0, 0)

    def pair_body(p, carry):
        c = p * 2
        start(c + 1, 1)
        finish(c, 0)
        start(c + 2, 0)
        finish(c + 1, 1)
        return carry

    lax.fori_loop(0, EC // 2 - 1, pair_body, 0)
    c = EC - 2
    start(c + 1, 1)
    finish(c, 0)
    finish(c + 1, 1)


def _sc_node_body(v_hbm, a_hbm, hu_hbm, att_hbm, hu_out, attg_out,
                  vds, ads, hb0, hb1, hb2, ab0, ab1, ab2, sh_sem, sa_sem):
    cid = lax.axis_index("c")
    sid = lax.axis_index("s")
    wid = sid * 2 + cid

    pltpu.sync_copy(v_hbm.at[wid], vds)
    pltpu.sync_copy(a_hbm.at[wid], ads)
    hb = (hb0, hb1, hb2)
    ab = (ab0, ab1, ab2)
    for c in range(NC_NODE):
        pltpu.async_copy(hu_hbm.at[vds.at[c]], hb[c], sh_sem)
        pltpu.async_copy(att_hbm.at[ads.at[c]], ab[c], sa_sem)
    for c in range(NC_NODE):
        pltpu.make_async_copy(hu_hbm.at[vds.at[c]], hb[c], sh_sem).wait()
        pltpu.sync_copy(hb[c], hu_out.at[pl.ds(wid * NWN + c * CH, CH)])
        pltpu.make_async_copy(att_hbm.at[ads.at[c]], ab[c], sa_sem).wait()
        pltpu.sync_copy(ab[c], attg_out.at[pl.ds(wid * NWN + c * CH, CH)])


def _tc_msg_body(pre_ref, pre2_ref, rel_ref, eg_ref, rt, qm, w2, b2, out_ref):
    bf = jnp.bfloat16
    ohr = (rel_ref[...] == lax.broadcasted_iota(jnp.int32, (1, 512), 1)
           ).astype(bf)
    ohe = (eg_ref[...] == lax.broadcasted_iota(jnp.int32, (1, B), 1)
           ).astype(bf)
    x = (pre_ref[...] + pre2_ref[...]
         + jnp.dot(ohr, rt[...].astype(bf), preferred_element_type=jnp.float32)
         + jnp.dot(ohe, qm[...].astype(bf), preferred_element_type=jnp.float32))
    y = jnp.where(x >= 0, x, 0.01 * x)
    z = jnp.dot(y.astype(bf), w2[...].astype(bf),
                preferred_element_type=jnp.float32) + b2[...]
    out_ref[...] = jnp.tanh(z)


def _sc_scatter_body(msg_hbm, js_hbm, seg_out, cnt_out,
                     msgbuf, msgbuf2, jds, ones1, zc,
                     sh_seg, sh_cnt, sm0, sm1):
    cid = lax.axis_index("c")
    sid = lax.axis_index("s")
    wid = sid * 2 + cid
    zero16 = jnp.zeros((16,), jnp.float32)
    one16 = jnp.ones((16,), jnp.float32)

    def zrow(r, rc):
        for k in range(8):
            msgbuf[r, pl.ds(k * 16, 16)] = zero16
        return rc

    lax.fori_loop(0, CH, zrow, 0)

    def fill1(r, rc):
        ones1[pl.ds(r * 16, 16)] = one16
        zc[pl.ds(r * 16, 16)] = zero16
        zc[pl.ds((r + 8) * 16, 16)] = zero16
        zc[pl.ds((r + 16) * 16, 16)] = zero16
        zc[pl.ds((r + 24) * 16, 16)] = zero16
        zc[pl.ds((r + 32) * 16, 16)] = zero16
        return rc

    lax.fori_loop(0, 8, fill1, 0)

    def zseg(t, rc):
        pltpu.sync_copy(msgbuf, sh_seg.at[pl.ds(sid * 640 + t * CH, CH)])
        return rc

    lax.fori_loop(0, 5, zseg, 0)
    pltpu.sync_copy(zc, sh_cnt.at[pl.ds(sid * 640, 640)])

    plsc.subcore_barrier()

    pltpu.sync_copy(js_hbm.at[wid], jds)

    mb = (msgbuf, msgbuf2)
    sm = (sm0, sm1)

    def start(c, ph):
        pltpu.async_copy(msg_hbm.at[pl.ds(wid * EW + c * CH, CH)], mb[ph], sm[ph])

    def finish(c, ph):
        pltpu.make_async_copy(msg_hbm.at[pl.ds(wid * EW + c * CH, CH)],
                              mb[ph], sm[ph]).wait()
        pltpu.sync_copy(mb[ph], sh_seg.at[jds.at[c]], add=True)
        pltpu.sync_copy(ones1, sh_cnt.at[jds.at[c]], add=True)

    start(0, 0)

    def pair_body(p, rc):
        c = p * 2
        start(c + 1, 1)
        finish(c, 0)
        start(c + 2, 0)
        finish(c + 1, 1)
        return rc

    lax.fori_loop(0, EC // 2 - 1, pair_body, 0)
    c2 = EC - 2
    start(c2 + 1, 1)
    finish(c2, 0)
    finish(c2 + 1, 1)

    plsc.subcore_barrier()

    pltpu.sync_copy(sh_seg.at[pl.ds(sid * 640, 640)],
                    seg_out.at[cid, pl.ds(sid * 640, 640)])
    pltpu.sync_copy(sh_cnt.at[pl.ds(sid * 640, 640)],
                    cnt_out.at[cid, pl.ds(sid * 640, 640)])


def _tc_final_body(seg_ref, cnt0_ref, cnt1_ref, hid_ref, hu_ref, att_ref,
                   meg_ref, wh1a, wh1b, wic, qh1, wh2, bh2, out_ref):
    seg = seg_ref[0] + seg_ref[1]
    cnt = cnt0_ref[...] + cnt1_ref[...]
    ma = seg * lax.rsqrt(jnp.maximum(cnt, 1.0))
    hid = hid_ref[...]
    hu = att_ref[...] * hu_ref[...]
    oh = (meg_ref[...] == lax.broadcasted_iota(jnp.int32, (1, B), 1)
          ).astype(jnp.float32)
    pre2 = (jnp.dot(ma, wh1a[...], preferred_element_type=jnp.float32)
            + jnp.dot(hid, wh1b[...], preferred_element_type=jnp.float32)
            + jnp.dot(hu, wic[...], preferred_element_type=jnp.float32)
            + jnp.dot(oh, qh1[...], preferred_element_type=jnp.float32))
    y = jnp.where(pre2 >= 0, pre2, 0.01 * pre2)
    out_ref[...] = hid + jnp.tanh(
        jnp.dot(y, wh2[...], preferred_element_type=jnp.float32) + bh2[...])


def _full(shape):
    return pl.BlockSpec(shape, lambda i: (0,) * len(shape))


def kernel(hidden, seen_edges, memorized_nodes, node_attention, hidden_uncon,
           query_head_emb, query_rel_emb, rel_table,
           W_m1, b_m1, W_m2, b_m2, W_h1, b_h1, W_h2, b_h2, W_int):
    f32 = jnp.float32

    eg = seen_edges[:, 0]
    rel = seen_edges[:, 3]
    jcol = seen_edges[:, 5]
    vi_row = seen_edges[:, 6]

    def pad_idx(x, value):
        return jnp.pad(x, (0, EPAD - E), constant_values=value).reshape(NW, EC, CH)

    vi_g = pad_idx(vi_row, 0)
    j_g = pad_idx(jcol, 0)
    j_s = pad_idx(jcol, NSEG - 1)

    mem_eg = memorized_nodes[:, 0]
    v = memorized_nodes[:, 1]
    v_p = jnp.pad(v, (0, NPAD - N_MEM)).reshape(NW, NC_NODE, CH)
    aflat = mem_eg * N_ENT + v
    a_p = jnp.pad(aflat, (0, NPAD - N_MEM)).reshape(NW, NC_NODE, CH)
    att_flat = node_attention.reshape(B * N_ENT)
    hu_tab = hidden_uncon.reshape(N_ENT, D)

    rel_pad = jnp.pad(rel_table, ((0, 512 - N_REL), (0, 0)))
    bm1 = b_m1.reshape(1, D)
    bm2 = b_m2.reshape(1, D)
    bh1 = b_h1.reshape(1, D)
    bh2 = b_h2.reshape(1, D)
    meg2 = jnp.pad(mem_eg, (0, NSEG - N_MEM)).reshape(NSEG, 1)

    hid_pad = jnp.pad(hidden, ((0, NSEG - N_MEM), (0, 0)))
    BL1 = 1024
    h1, h2, rt, qm1, qh1, wic = pl.pallas_call(
        _tc_pre_body,
        grid=(NSEG // BL1,),
        in_specs=[
            pl.BlockSpec((BL1, D), lambda i: (i, 0)),
            _full((D, D)), _full((D, D)), _full((512, D)), _full((D, D)),
            _full((B, D)), _full((B, D)), _full((D, D)), _full((D, D)),
            _full((1, D)), _full((D, D)), _full((D, D)), _full((1, D)),
            _full((D, D)), _full((D, D)),
        ],
        out_specs=[
            pl.BlockSpec((BL1, D), lambda i: (i, 0)),
            pl.BlockSpec((BL1, D), lambda i: (i, 0)),
            _full((512, D)), _full((B, D)), _full((B, D)), _full((D, D)),
        ],
        out_shape=[
            jax.ShapeDtypeStruct((NSEG, D), f32),
            jax.ShapeDtypeStruct((NSEG, D), f32),
            jax.ShapeDtypeStruct((512, D), f32),
            jax.ShapeDtypeStruct((B, D), f32),
            jax.ShapeDtypeStruct((B, D), f32),
            jax.ShapeDtypeStruct((D, D), f32),
        ],
    )(hid_pad, W_m1[0:D], W_m1[2 * D:3 * D], rel_pad, W_m1[D:2 * D],
      query_head_emb, query_rel_emb, W_m1[3 * D:4 * D], W_m1[4 * D:5 * D], bm1,
      W_h1[3 * D:4 * D], W_h1[4 * D:5 * D], bh1, W_int, W_h1[2 * D:3 * D])

    mesh = plsc.VectorSubcoreMesh(core_axis_name="c", subcore_axis_name="s")

    def tab_gather(tab, idx):
        return pl.kernel(
            _sc_tab_gather_body,
            out_type=jax.ShapeDtypeStruct((EPAD, D), f32),
            mesh=mesh,
            scratch_types=[
                pltpu.VMEM((EC, CH), jnp.int32),
                pltpu.VMEM((CH, D), f32),
                pltpu.VMEM((CH, D), f32),
                pltpu.VMEM_SHARED((NSEG, D), f32),
                pltpu.SemaphoreType.DMA,
                pltpu.SemaphoreType.DMA,
            ],
        )(tab, idx)

    pre = tab_gather(h1, vi_g)
    pre2 = tab_gather(h2, j_g)

    hu_g, att_g = pl.kernel(
        _sc_node_body,
        out_type=[
            jax.ShapeDtypeStruct((NPAD, D), f32),
            jax.ShapeDtypeStruct((NPAD,), f32),
        ],
        mesh=mesh,
        scratch_types=[
            pltpu.VMEM((NC_NODE, CH), jnp.int32),
            pltpu.VMEM((NC_NODE, CH), jnp.int32),
            pltpu.VMEM((CH, D), f32),
            pltpu.VMEM((CH, D), f32),
            pltpu.VMEM((CH, D), f32),
            pltpu.VMEM((CH,), f32),
            pltpu.VMEM((CH,), f32),
            pltpu.VMEM((CH,), f32),
            pltpu.SemaphoreType.DMA,
            pltpu.SemaphoreType.DMA,
        ],
    )(v_p, a_p, hu_tab, att_flat)

    BL3 = 2048
    rel_col = jnp.pad(rel, (0, EPAD - E)).reshape(EPAD, 1)
    eg_col = jnp.pad(eg, (0, EPAD - E)).reshape(EPAD, 1)
    msg = pl.pallas_call(
        _tc_msg_body,
        grid=(EPAD // BL3,),
        in_specs=[
            pl.BlockSpec((BL3, D), lambda i: (i, 0)),
            pl.BlockSpec((BL3, D), lambda i: (i, 0)),
            pl.BlockSpec((BL3, 1), lambda i: (i, 0)),
            pl.BlockSpec((BL3, 1), lambda i: (i, 0)),
            _full((512, D)), _full((B, D)),
            _full((D, D)), _full((1, D)),
        ],
        out_specs=pl.BlockSpec((BL3, D), lambda i: (i, 0)),
        out_shape=jax.ShapeDtypeStruct((EPAD, D), f32),
    )(pre, pre2, rel_col, eg_col, rt, qm1, W_m2, bm2)

    seg, cntp = pl.kernel(
        _sc_scatter_body,
        out_type=[
            jax.ShapeDtypeStruct((2, NSEG, D), f32),
            jax.ShapeDtypeStruct((2, NSEG), f32),
        ],
        mesh=mesh,
        scratch_types=[
            pltpu.VMEM((CH, D), f32),
            pltpu.VMEM((CH, D), f32),
            pltpu.VMEM((EC, CH), jnp.int32),
            pltpu.VMEM((CH,), f32),
            pltpu.VMEM((640,), f32),
            pltpu.VMEM_SHARED((NSEG, D), f32),
            pltpu.VMEM_SHARED((NSEG,), f32),
            pltpu.SemaphoreType.DMA,
            pltpu.SemaphoreType.DMA,
        ],
    )(msg, j_s)

    BL5 = 1024
    nblk = NSEG // BL5
    cnt_flat = cntp.reshape(2 * NSEG, 1)
    out = pl.pallas_call(
        _tc_final_body,
        grid=(nblk,),
        in_specs=[
            pl.BlockSpec((2, BL5, D), lambda i: (0, i, 0)),
            pl.BlockSpec((BL5, 1), lambda i: (i, 0)),
            pl.BlockSpec((BL5, 1), lambda i: (i + nblk, 0)),
            pl.BlockSpec((BL5, D), lambda i: (i, 0)),
            pl.BlockSpec((BL5, D), lambda i: (i, 0)),
            pl.BlockSpec((BL5, 1), lambda i: (i, 0)),
            pl.BlockSpec((BL5, 1), lambda i: (i, 0)),
            _full((D, D)), _full((D, D)), _full((D, D)),
            _full((B, D)), _full((D, D)), _full((1, D)),
        ],
        out_specs=pl.BlockSpec((BL5, D), lambda i: (i, 0)),
        out_shape=jax.ShapeDtypeStruct((NSEG, D), f32),
    )(seg, cnt_flat, cnt_flat, hid_pad, hu_g, att_g.reshape(-1, 1), meg2,
      W_h1[0:D], W_h1[D:2 * D], wic, qh1, W_h2, bh2)

    return out[:N_MEM]


# speedup vs baseline: 1.3003x; 1.0434x over previous
import functools

import jax
import jax.numpy as jnp
from jax import lax
from jax.experimental import pallas as pl
from jax.experimental.pallas import tpu as pltpu
from jax.experimental.pallas import tpu_sc as plsc

N_MEM = 10000
E = 160000
B = 64
N_ENT = 100000
D = 128
N_REL = 500

NW = 32
CH = 128
EC = 40
EW = CH * EC
EPAD = NW * EW
NSEG = 10240
NC_NODE = 3
NWN = CH * NC_NODE
NPAD = NW * NWN


def _tc_pre_body(hid, wm1a, wm1c, relp, wm1b, qh, qr, wm1d, wm1e, bm1,
                 wh1d, wh1e, bh1, wint, wh1c,
                 h1_o, h2_o, rt_o, qm_o, qhn_o, wic_o):
    h = hid[...]
    h1_o[...] = jnp.dot(h, wm1a[...], preferred_element_type=jnp.float32)
    h2_o[...] = jnp.dot(h, wm1c[...], preferred_element_type=jnp.float32)

    @pl.when(pl.program_id(0) == 0)
    def _():
        rt_o[...] = jnp.dot(relp[...], wm1b[...], preferred_element_type=jnp.float32)
        qhv = qh[...]
        qrv = qr[...]
        qm_o[...] = (jnp.dot(qhv, wm1d[...], preferred_element_type=jnp.float32)
                     + jnp.dot(qrv, wm1e[...], preferred_element_type=jnp.float32)
                     + bm1[...])
        qhn_o[...] = (jnp.dot(qhv, wh1d[...], preferred_element_type=jnp.float32)
                      + jnp.dot(qrv, wh1e[...], preferred_element_type=jnp.float32)
                      + bh1[...])
        wic_o[...] = jnp.dot(wint[...], wh1c[...], preferred_element_type=jnp.float32)


ROWS_W = 640


def _sc_tab_gather_body(tab_hbm, idx_hbm, out_hbm,
                        idx_v, ba, bb, sh_tab, sa, sb):
    cid = lax.axis_index("c")
    sid = lax.axis_index("s")
    wid = sid * 2 + cid

    pltpu.sync_copy(idx_hbm.at[wid], idx_v)
    pltpu.sync_copy(tab_hbm.at[pl.ds(sid * ROWS_W, ROWS_W)],
                    sh_tab.at[pl.ds(sid * ROWS_W, ROWS_W)])
    plsc.subcore_barrier()

    gb = (ba, bb)
    sg = (sa, sb)

    def start(c, ph):
        pltpu.async_copy(sh_tab.at[idx_v.at[c]], gb[ph], sg[ph])

    def finish(c, ph):
        pltpu.make_async_copy(sh_tab.at[idx_v.at[c]], gb[ph], sg[ph]).wait()
        pltpu.sync_copy(gb[ph], out_hbm.at[pl.ds(wid * EW + c * CH, CH)])

    start(0, 0)

    def pair_body(p, carry):
        c = p * 2
        start(c + 1, 1)
        finish(c, 0)
        start(c + 2, 0)
        finish(c + 1, 1)
        return carry

    lax.fori_loop(0, EC // 2 - 1, pair_body, 0)
    c = EC - 2
    start(c + 1, 1)
    finish(c, 0)
    finish(c + 1, 1)


def _sc_node_body(v_hbm, a_hbm, hu_hbm, att_hbm, hu_out, attg_out,
                  vds, ads, hb0, hb1, hb2, ab0, ab1, ab2, sh_sem, sa_sem):
    cid = lax.axis_index("c")
    sid = lax.axis_index("s")
    wid = sid * 2 + cid

    pltpu.sync_copy(v_hbm.at[wid], vds)
    pltpu.sync_copy(a_hbm.at[wid], ads)
    hb = (hb0, hb1, hb2)
    ab = (ab0, ab1, ab2)
    for c in range(NC_NODE):
        pltpu.async_copy(hu_hbm.at[vds.at[c]], hb[c], sh_sem)
        pltpu.async_copy(att_hbm.at[ads.at[c]], ab[c], sa_sem)
    for c in range(NC_NODE):
        pltpu.make_async_copy(hu_hbm.at[vds.at[c]], hb[c], sh_sem).wait()
        pltpu.sync_copy(hb[c], hu_out.at[pl.ds(wid * NWN + c * CH, CH)])
        pltpu.make_async_copy(att_hbm.at[ads.at[c]], ab[c], sa_sem).wait()
        pltpu.sync_copy(ab[c], attg_out.at[pl.ds(wid * NWN + c * CH, CH)])


def _tc_msg_body(pre_ref, pre2_ref, rel_ref, eg_ref, rt, qm, w2, b2, out_ref):
    bf = jnp.bfloat16
    ohr = (rel_ref[...] == lax.broadcasted_iota(jnp.int32, (1, 512), 1)
           ).astype(bf)
    ohe = (eg_ref[...] == lax.broadcasted_iota(jnp.int32, (1, B), 1)
           ).astype(bf)
    x = (pre_ref[...] + pre2_ref[...]
         + jnp.dot(ohr, rt[...].astype(bf), preferred_element_type=jnp.float32)
         + jnp.dot(ohe, qm[...].astype(bf), preferred_element_type=jnp.float32))
    y = jnp.where(x >= 0, x, 0.01 * x)
    z = jnp.dot(y.astype(bf), w2[...].astype(bf),
                preferred_element_type=jnp.float32) + b2[...]
    out_ref[...] = jnp.tanh(z)


def _sc_scatter_body(msg_hbm, js_hbm, seg_out, cnt_out,
                     msgbuf, msgbuf2, jds, ones1, zc,
                     sh_seg, sh_cnt, sm0, sm1):
    cid = lax.axis_index("c")
    sid = lax.axis_index("s")
    wid = sid * 2 + cid
    zero16 = jnp.zeros((16,), jnp.float32)
    one16 = jnp.ones((16,), jnp.float32)

    def zrow(r, rc):
        for k in range(8):
            msgbuf[r, pl.ds(k * 16, 16)] = zero16
        return rc

    lax.fori_loop(0, CH, zrow, 0)

    def fill1(r, rc):
        ones1[pl.ds(r * 16, 16)] = one16
        zc[pl.ds(r * 16, 16)] = zero16
        zc[pl.ds((r + 8) * 16, 16)] = zero16
        zc[pl.ds((r + 16) * 16, 16)] = zero16
        zc[pl.ds((r + 24) * 16, 16)] = zero16
        zc[pl.ds((r + 32) * 16, 16)] = zero16
        return rc

    lax.fori_loop(0, 8, fill1, 0)

    def zseg(t, rc):
        pltpu.sync_copy(msgbuf, sh_seg.at[pl.ds(sid * 640 + t * CH, CH)])
        return rc

    lax.fori_loop(0, 5, zseg, 0)
    pltpu.sync_copy(zc, sh_cnt.at[pl.ds(sid * 640, 640)])

    plsc.subcore_barrier()

    pltpu.sync_copy(js_hbm.at[wid], jds)

    mb = (msgbuf, msgbuf2)
    sm = (sm0, sm1)

    def start(c, ph):
        pltpu.async_copy(msg_hbm.at[pl.ds(wid * EW + c * CH, CH)], mb[ph], sm[ph])

    def finish(c, ph):
        pltpu.make_async_copy(msg_hbm.at[pl.ds(wid * EW + c * CH, CH)],
                              mb[ph], sm[ph]).wait()
        pltpu.sync_copy(mb[ph], sh_seg.at[jds.at[c]], add=True)
        pltpu.sync_copy(ones1, sh_cnt.at[jds.at[c]], add=True)

    start(0, 0)

    def pair_body(p, rc):
        c = p * 2
        start(c + 1, 1)
        finish(c, 0)
        start(c + 2, 0)
        finish(c + 1, 1)
        return rc

    lax.fori_loop(0, EC // 2 - 1, pair_body, 0)
    c2 = EC - 2
    start(c2 + 1, 1)
    finish(c2, 0)
    finish(c2 + 1, 1)

    plsc.subcore_barrier()

    pltpu.sync_copy(sh_seg.at[pl.ds(sid * 640, 640)],
                    seg_out.at[cid, pl.ds(sid * 640, 640)])
    pltpu.sync_copy(sh_cnt.at[pl.ds(sid * 640, 640)],
                    cnt_out.at[cid, pl.ds(sid * 640, 640)])


def _tc_final_body(seg_ref, cnt0_ref, cnt1_ref, hid_ref, hu_ref, att_ref,
                   meg_ref, wh1a, wh1b, wic, qh1, wh2, bh2, out_ref):
    seg = seg_ref[0] + seg_ref[1]
    cnt = cnt0_ref[...] + cnt1_ref[...]
    ma = seg * lax.rsqrt(jnp.maximum(cnt, 1.0))
    hid = hid_ref[...]
    hu = att_ref[...] * hu_ref[...]
    oh = (meg_ref[...] == lax.broadcasted_iota(jnp.int32, (1, B), 1)
          ).astype(jnp.float32)
    pre2 = (jnp.dot(ma, wh1a[...], preferred_element_type=jnp.float32)
            + jnp.dot(hid, wh1b[...], preferred_element_type=jnp.float32)
            + jnp.dot(hu, wic[...], preferred_element_type=jnp.float32)
            + jnp.dot(oh, qh1[...], preferred_element_type=jnp.float32))
    y = jnp.where(pre2 >= 0, pre2, 0.01 * pre2)
    out_ref[...] = hid + jnp.tanh(
        jnp.dot(y, wh2[...], preferred_element_type=jnp.float32) + bh2[...])


def _full(shape):
    return pl.BlockSpec(shape, lambda i: (0,) * len(shape))


def kernel(hidden, seen_edges, memorized_nodes, node_attention, hidden_uncon,
           query_head_emb, query_rel_emb, rel_table,
           W_m1, b_m1, W_m2, b_m2, W_h1, b_h1, W_h2, b_h2, W_int):
    f32 = jnp.float32

    eg = seen_edges[:, 0]
    rel = seen_edges[:, 3]
    jcol = seen_edges[:, 5]
    vi_row = seen_edges[:, 6]

    def pad_idx(x, value):
        return jnp.pad(x, (0, EPAD - E), constant_values=value).reshape(NW, EC, CH)

    vi_g = pad_idx(vi_row, 0)
    j_g = pad_idx(jcol, 0)
    j_s = pad_idx(jcol, NSEG - 1)

    mem_eg = memorized_nodes[:, 0]
    v = memorized_nodes[:, 1]
    v_p = jnp.pad(v, (0, NPAD - N_MEM)).reshape(NW, NC_NODE, CH)
    aflat = mem_eg * N_ENT + v
    a_p = jnp.pad(aflat, (0, NPAD - N_MEM)).reshape(NW, NC_NODE, CH)
    att_flat = node_attention.reshape(B * N_ENT)
    hu_tab = hidden_uncon.reshape(N_ENT, D)

    rel_pad = jnp.pad(rel_table, ((0, 512 - N_REL), (0, 0)))
    bm1 = b_m1.reshape(1, D)
    bm2 = b_m2.reshape(1, D)
    bh1 = b_h1.reshape(1, D)
    bh2 = b_h2.reshape(1, D)
    meg2 = jnp.pad(mem_eg, (0, NSEG - N_MEM)).reshape(NSEG, 1)

    hid_pad = jnp.pad(hidden, ((0, NSEG - N_MEM), (0, 0)))
    BL1 = 1024
    h1, h2, rt, qm1, qh1, wic = pl.pallas_call(
        _tc_pre_body,
        grid=(NSEG // BL1,),
        in_specs=[
            pl.BlockSpec((BL1, D), lambda i: (i, 0)),
            _full((D, D)), _full((D, D)), _full((512, D)), _full((D, D)),
            _full((B, D)), _full((B, D)), _full((D, D)), _full((D, D)),
            _full((1, D)), _full((D, D)), _full((D, D)), _full((1, D)),
            _full((D, D)), _full((D, D)),
        ],
        out_specs=[
            pl.BlockSpec((BL1, D), lambda i: (i, 0)),
            pl.BlockSpec((BL1, D), lambda i: (i, 0)),
            _full((512, D)), _full((B, D)), _full((B, D)), _full((D, D)),
        ],
        out_shape=[
            jax.ShapeDtypeStruct((NSEG, D), f32),
            jax.ShapeDtypeStruct((NSEG, D), f32),
            jax.ShapeDtypeStruct((512, D), f32),
            jax.ShapeDtypeStruct((B, D), f32),
            jax.ShapeDtypeStruct((B, D), f32),
            jax.ShapeDtypeStruct((D, D), f32),
        ],
    )(hid_pad, W_m1[0:D], W_m1[2 * D:3 * D], rel_pad, W_m1[D:2 * D],
      query_head_emb, query_rel_emb, W_m1[3 * D:4 * D], W_m1[4 * D:5 * D], bm1,
      W_h1[3 * D:4 * D], W_h1[4 * D:5 * D], bh1, W_int, W_h1[2 * D:3 * D])

    mesh = plsc.VectorSubcoreMesh(core_axis_name="c", subcore_axis_name="s")

    def tab_gather(tab, idx):
        return pl.kernel(
            _sc_tab_gather_body,
            out_type=jax.ShapeDtypeStruct((EPAD, D), f32),
            mesh=mesh,
            scratch_types=[
                pltpu.VMEM((EC, CH), jnp.int32),
                pltpu.VMEM((CH, D), f32),
                pltpu.VMEM((CH, D), f32),
                pltpu.VMEM_SHARED((NSEG, D), f32),
                pltpu.SemaphoreType.DMA,
                pltpu.SemaphoreType.DMA,
            ],
        )(tab, idx)

    pre = tab_gather(h1, vi_g)
    pre2 = tab_gather(h2, j_g)

    hu_g, att_g = pl.kernel(
        _sc_node_body,
        out_type=[
            jax.ShapeDtypeStruct((NPAD, D), f32),
            jax.ShapeDtypeStruct((NPAD,), f32),
        ],
        mesh=mesh,
        scratch_types=[
            pltpu.VMEM((NC_NODE, CH), jnp.int32),
            pltpu.VMEM((NC_NODE, CH), jnp.int32),
            pltpu.VMEM((CH, D), f32),
            pltpu.VMEM((CH, D), f32),
            pltpu.VMEM((CH, D), f32),
            pltpu.VMEM((CH,), f32),
            pltpu.VMEM((CH,), f32),
            pltpu.VMEM((CH,), f32),
            pltpu.SemaphoreType.DMA,
            pltpu.SemaphoreType.DMA,
        ],
    )(v_p, a_p, hu_tab, att_flat)

    BL3 = 4096
    rel_col = jnp.pad(rel, (0, EPAD - E)).reshape(EPAD, 1)
    eg_col = jnp.pad(eg, (0, EPAD - E)).reshape(EPAD, 1)
    msg = pl.pallas_call(
        _tc_msg_body,
        grid=(EPAD // BL3,),
        in_specs=[
            pl.BlockSpec((BL3, D), lambda i: (i, 0)),
            pl.BlockSpec((BL3, D), lambda i: (i, 0)),
            pl.BlockSpec((BL3, 1), lambda i: (i, 0)),
            pl.BlockSpec((BL3, 1), lambda i: (i, 0)),
            _full((512, D)), _full((B, D)),
            _full((D, D)), _full((1, D)),
        ],
        out_specs=pl.BlockSpec((BL3, D), lambda i: (i, 0)),
        out_shape=jax.ShapeDtypeStruct((EPAD, D), f32),
    )(pre, pre2, rel_col, eg_col, rt, qm1, W_m2, bm2)

    seg, cntp = pl.kernel(
        _sc_scatter_body,
        out_type=[
            jax.ShapeDtypeStruct((2, NSEG, D), f32),
            jax.ShapeDtypeStruct((2, NSEG), f32),
        ],
        mesh=mesh,
        scratch_types=[
            pltpu.VMEM((CH, D), f32),
            pltpu.VMEM((CH, D), f32),
            pltpu.VMEM((EC, CH), jnp.int32),
            pltpu.VMEM((CH,), f32),
            pltpu.VMEM((640,), f32),
            pltpu.VMEM_SHARED((NSEG, D), f32),
            pltpu.VMEM_SHARED((NSEG,), f32),
            pltpu.SemaphoreType.DMA,
            pltpu.SemaphoreType.DMA,
        ],
    )(msg, j_s)

    BL5 = 1024
    nblk = NSEG // BL5
    cnt_flat = cntp.reshape(2 * NSEG, 1)
    out = pl.pallas_call(
        _tc_final_body,
        grid=(nblk,),
        in_specs=[
            pl.BlockSpec((2, BL5, D), lambda i: (0, i, 0)),
            pl.BlockSpec((BL5, 1), lambda i: (i, 0)),
            pl.BlockSpec((BL5, 1), lambda i: (i + nblk, 0)),
            pl.BlockSpec((BL5, D), lambda i: (i, 0)),
            pl.BlockSpec((BL5, D), lambda i: (i, 0)),
            pl.BlockSpec((BL5, 1), lambda i: (i, 0)),
            pl.BlockSpec((BL5, 1), lambda i: (i, 0)),
            _full((D, D)), _full((D, D)), _full((D, D)),
            _full((B, D)), _full((D, D)), _full((1, D)),
        ],
        out_specs=pl.BlockSpec((BL5, D), lambda i: (i, 0)),
        out_shape=jax.ShapeDtypeStruct((NSEG, D), f32),
    )(seg, cnt_flat, cnt_flat, hid_pad, hu_g, att_g.reshape(-1, 1), meg2,
      W_h1[0:D], W_h1[D:2 * D], wic, qh1, W_h2, bh2)

    return out[:N_MEM]


# speedup vs baseline: 1.3109x; 1.0081x over previous
import functools

import jax
import jax.numpy as jnp
from jax import lax
from jax.experimental import pallas as pl
from jax.experimental.pallas import tpu as pltpu
from jax.experimental.pallas import tpu_sc as plsc

N_MEM = 10000
E = 160000
B = 64
N_ENT = 100000
D = 128
N_REL = 500

NW = 32
CH = 128
EC = 40
EW = CH * EC
EPAD = NW * EW
NSEG = 10240
NC_NODE = 3
NWN = CH * NC_NODE
NPAD = NW * NWN


def _tc_pre_body(hid, wm1a, wm1c, relp, wm1b, qh, qr, wm1d, wm1e, bm1,
                 wh1d, wh1e, bh1, wint, wh1c,
                 h1_o, h2_o, rt_o, qm_o, qhn_o, wic_o):
    h = hid[...]
    h1_o[...] = jnp.dot(h, wm1a[...], preferred_element_type=jnp.float32)
    h2_o[...] = jnp.dot(h, wm1c[...], preferred_element_type=jnp.float32)

    @pl.when(pl.program_id(0) == 0)
    def _():
        rt_o[...] = jnp.dot(relp[...], wm1b[...], preferred_element_type=jnp.float32)
        qhv = qh[...]
        qrv = qr[...]
        qm_o[...] = (jnp.dot(qhv, wm1d[...], preferred_element_type=jnp.float32)
                     + jnp.dot(qrv, wm1e[...], preferred_element_type=jnp.float32)
                     + bm1[...])
        qhn_o[...] = (jnp.dot(qhv, wh1d[...], preferred_element_type=jnp.float32)
                      + jnp.dot(qrv, wh1e[...], preferred_element_type=jnp.float32)
                      + bh1[...])
        wic_o[...] = jnp.dot(wint[...], wh1c[...], preferred_element_type=jnp.float32)


ROWS_W = 640


def _sc_tab_gather_body(tab_hbm, idx_hbm, out_hbm,
                        idx_v, ba, bb, sh_tab, sa, sb):
    cid = lax.axis_index("c")
    sid = lax.axis_index("s")
    wid = sid * 2 + cid

    pltpu.sync_copy(idx_hbm.at[wid], idx_v)
    pltpu.sync_copy(tab_hbm.at[pl.ds(sid * ROWS_W, ROWS_W)],
                    sh_tab.at[pl.ds(sid * ROWS_W, ROWS_W)])
    plsc.subcore_barrier()

    gb = (ba, bb)
    sg = (sa, sb)

    def start(c, ph):
        pltpu.async_copy(sh_tab.at[idx_v.at[c]], gb[ph], sg[ph])

    def finish(c, ph):
        pltpu.make_async_copy(sh_tab.at[idx_v.at[c]], gb[ph], sg[ph]).wait()
        pltpu.sync_copy(gb[ph], out_hbm.at[pl.ds(wid * EW + c * CH, CH)])

    start(0, 0)

    def pair_body(p, carry):
        c = p * 2
        start(c + 1, 1)
        finish(c, 0)
        start(c + 2, 0)
        finish(c + 1, 1)
        return carry

    lax.fori_loop(0, EC // 2 - 1, pair_body, 0)
    c = EC - 2
    start(c + 1, 1)
    finish(c, 0)
    finish(c + 1, 1)


def _sc_node_body(v_hbm, a_hbm, hu_hbm, att_hbm, hu_out, attg_out,
                  vds, ads, hb0, hb1, hb2, ab0, ab1, ab2, sh_sem, sa_sem):
    cid = lax.axis_index("c")
    sid = lax.axis_index("s")
    wid = sid * 2 + cid

    pltpu.sync_copy(v_hbm.at[wid], vds)
    pltpu.sync_copy(a_hbm.at[wid], ads)
    hb = (hb0, hb1, hb2)
    ab = (ab0, ab1, ab2)
    for c in range(NC_NODE):
        pltpu.async_copy(hu_hbm.at[vds.at[c]], hb[c], sh_sem)
        pltpu.async_copy(att_hbm.at[ads.at[c]], ab[c], sa_sem)
    for c in range(NC_NODE):
        pltpu.make_async_copy(hu_hbm.at[vds.at[c]], hb[c], sh_sem).wait()
        pltpu.sync_copy(hb[c], hu_out.at[pl.ds(wid * NWN + c * CH, CH)])
        pltpu.make_async_copy(att_hbm.at[ads.at[c]], ab[c], sa_sem).wait()
        pltpu.sync_copy(ab[c], attg_out.at[pl.ds(wid * NWN + c * CH, CH)])


def _tc_msg_body(pre_ref, pre2_ref, rel_ref, eg_ref, rt, qm, w2, b2, out_ref):
    bf = jnp.bfloat16
    ohr = (rel_ref[...] == lax.broadcasted_iota(jnp.int32, (1, 512), 1)
           ).astype(bf)
    ohe = (eg_ref[...] == lax.broadcasted_iota(jnp.int32, (1, B), 1)
           ).astype(bf)
    x = (pre_ref[...] + pre2_ref[...]
         + jnp.dot(ohr, rt[...].astype(bf), preferred_element_type=jnp.float32)
         + jnp.dot(ohe, qm[...].astype(bf), preferred_element_type=jnp.float32))
    y = jnp.where(x >= 0, x, 0.01 * x)
    z = jnp.dot(y.astype(bf), w2[...].astype(bf),
                preferred_element_type=jnp.float32) + b2[...]
    out_ref[...] = jnp.tanh(z)


def _sc_scatter_body(msg_hbm, js_hbm, seg_out, cnt_out,
                     msgbuf, msgbuf2, jds, ones1, zc,
                     sh_seg, sh_cnt, sm0, sm1):
    cid = lax.axis_index("c")
    sid = lax.axis_index("s")
    wid = sid * 2 + cid
    zero16 = jnp.zeros((16,), jnp.float32)
    one16 = jnp.ones((16,), jnp.float32)

    def zrow(r, rc):
        for k in range(8):
            msgbuf[r, pl.ds(k * 16, 16)] = zero16
        return rc

    lax.fori_loop(0, CH, zrow, 0)

    def fill1(r, rc):
        ones1[pl.ds(r * 16, 16)] = one16
        zc[pl.ds(r * 16, 16)] = zero16
        zc[pl.ds((r + 8) * 16, 16)] = zero16
        zc[pl.ds((r + 16) * 16, 16)] = zero16
        zc[pl.ds((r + 24) * 16, 16)] = zero16
        zc[pl.ds((r + 32) * 16, 16)] = zero16
        return rc

    lax.fori_loop(0, 8, fill1, 0)

    def zseg(t, rc):
        pltpu.sync_copy(msgbuf, sh_seg.at[pl.ds(sid * 640 + t * CH, CH)])
        return rc

    lax.fori_loop(0, 5, zseg, 0)
    pltpu.sync_copy(zc, sh_cnt.at[pl.ds(sid * 640, 640)])

    plsc.subcore_barrier()

    pltpu.sync_copy(js_hbm.at[wid], jds)

    mb = (msgbuf, msgbuf2)
    sm = (sm0, sm1)

    def start(c, ph):
        pltpu.async_copy(msg_hbm.at[pl.ds(wid * EW + c * CH, CH)], mb[ph], sm[ph])

    def finish(c, ph):
        pltpu.make_async_copy(msg_hbm.at[pl.ds(wid * EW + c * CH, CH)],
                              mb[ph], sm[ph]).wait()
        pltpu.sync_copy(mb[ph], sh_seg.at[jds.at[c]], add=True)
        pltpu.sync_copy(ones1, sh_cnt.at[jds.at[c]], add=True)

    start(0, 0)

    def pair_body(p, rc):
        c = p * 2
        start(c + 1, 1)
        finish(c, 0)
        start(c + 2, 0)
        finish(c + 1, 1)
        return rc

    lax.fori_loop(0, EC // 2 - 1, pair_body, 0)
    c2 = EC - 2
    start(c2 + 1, 1)
    finish(c2, 0)
    finish(c2 + 1, 1)

    plsc.subcore_barrier()

    pltpu.sync_copy(sh_seg.at[pl.ds(sid * 640, 640)],
                    seg_out.at[cid, pl.ds(sid * 640, 640)])
    pltpu.sync_copy(sh_cnt.at[pl.ds(sid * 640, 640)],
                    cnt_out.at[cid, pl.ds(sid * 640, 640)])


def _tc_final_body(seg_ref, cnt0_ref, cnt1_ref, hid_ref, hu_ref, att_ref,
                   meg_ref, wh1a, wh1b, wic, qh1, wh2, bh2, out_ref):
    seg = seg_ref[0] + seg_ref[1]
    cnt = cnt0_ref[...] + cnt1_ref[...]
    ma = seg * lax.rsqrt(jnp.maximum(cnt, 1.0))
    hid = hid_ref[...]
    hu = att_ref[...] * hu_ref[...]
    oh = (meg_ref[...] == lax.broadcasted_iota(jnp.int32, (1, B), 1)
          ).astype(jnp.float32)
    pre2 = (jnp.dot(ma, wh1a[...], preferred_element_type=jnp.float32)
            + jnp.dot(hid, wh1b[...], preferred_element_type=jnp.float32)
            + jnp.dot(hu, wic[...], preferred_element_type=jnp.float32)
            + jnp.dot(oh, qh1[...], preferred_element_type=jnp.float32))
    y = jnp.where(pre2 >= 0, pre2, 0.01 * pre2)
    out_ref[...] = hid + jnp.tanh(
        jnp.dot(y, wh2[...], preferred_element_type=jnp.float32) + bh2[...])


def _full(shape):
    return pl.BlockSpec(shape, lambda i: (0,) * len(shape))


def kernel(hidden, seen_edges, memorized_nodes, node_attention, hidden_uncon,
           query_head_emb, query_rel_emb, rel_table,
           W_m1, b_m1, W_m2, b_m2, W_h1, b_h1, W_h2, b_h2, W_int):
    f32 = jnp.float32

    eg = seen_edges[:, 0]
    rel = seen_edges[:, 3]
    jcol = seen_edges[:, 5]
    vi_row = seen_edges[:, 6]

    def pad_idx(x, value):
        return jnp.pad(x, (0, EPAD - E), constant_values=value).reshape(NW, EC, CH)

    vi_g = pad_idx(vi_row, 0)
    j_g = pad_idx(jcol, 0)
    j_s = pad_idx(jcol, NSEG - 1)

    mem_eg = memorized_nodes[:, 0]
    v = memorized_nodes[:, 1]
    v_p = jnp.pad(v, (0, NPAD - N_MEM)).reshape(NW, NC_NODE, CH)
    aflat = mem_eg * N_ENT + v
    a_p = jnp.pad(aflat, (0, NPAD - N_MEM)).reshape(NW, NC_NODE, CH)
    att_flat = node_attention.reshape(B * N_ENT)
    hu_tab = hidden_uncon.reshape(N_ENT, D)

    rel_pad = jnp.pad(rel_table, ((0, 512 - N_REL), (0, 0)))
    bm1 = b_m1.reshape(1, D)
    bm2 = b_m2.reshape(1, D)
    bh1 = b_h1.reshape(1, D)
    bh2 = b_h2.reshape(1, D)
    meg2 = jnp.pad(mem_eg, (0, NSEG - N_MEM)).reshape(NSEG, 1)

    hid_pad = jnp.pad(hidden, ((0, NSEG - N_MEM), (0, 0)))
    BL1 = 1024
    h1, h2, rt, qm1, qh1, wic = pl.pallas_call(
        _tc_pre_body,
        grid=(NSEG // BL1,),
        in_specs=[
            pl.BlockSpec((BL1, D), lambda i: (i, 0)),
            _full((D, D)), _full((D, D)), _full((512, D)), _full((D, D)),
            _full((B, D)), _full((B, D)), _full((D, D)), _full((D, D)),
            _full((1, D)), _full((D, D)), _full((D, D)), _full((1, D)),
            _full((D, D)), _full((D, D)),
        ],
        out_specs=[
            pl.BlockSpec((BL1, D), lambda i: (i, 0)),
            pl.BlockSpec((BL1, D), lambda i: (i, 0)),
            _full((512, D)), _full((B, D)), _full((B, D)), _full((D, D)),
        ],
        out_shape=[
            jax.ShapeDtypeStruct((NSEG, D), f32),
            jax.ShapeDtypeStruct((NSEG, D), f32),
            jax.ShapeDtypeStruct((512, D), f32),
            jax.ShapeDtypeStruct((B, D), f32),
            jax.ShapeDtypeStruct((B, D), f32),
            jax.ShapeDtypeStruct((D, D), f32),
        ],
    )(hid_pad, W_m1[0:D], W_m1[2 * D:3 * D], rel_pad, W_m1[D:2 * D],
      query_head_emb, query_rel_emb, W_m1[3 * D:4 * D], W_m1[4 * D:5 * D], bm1,
      W_h1[3 * D:4 * D], W_h1[4 * D:5 * D], bh1, W_int, W_h1[2 * D:3 * D])

    mesh = plsc.VectorSubcoreMesh(core_axis_name="c", subcore_axis_name="s")

    def tab_gather(tab, idx):
        return pl.kernel(
            _sc_tab_gather_body,
            out_type=jax.ShapeDtypeStruct((EPAD, D), f32),
            mesh=mesh,
            scratch_types=[
                pltpu.VMEM((EC, CH), jnp.int32),
                pltpu.VMEM((CH, D), f32),
                pltpu.VMEM((CH, D), f32),
                pltpu.VMEM_SHARED((NSEG, D), f32),
                pltpu.SemaphoreType.DMA,
                pltpu.SemaphoreType.DMA,
            ],
        )(tab, idx)

    pre = tab_gather(h1, vi_g)
    pre2 = tab_gather(h2, j_g)

    hu_g, att_g = pl.kernel(
        _sc_node_body,
        out_type=[
            jax.ShapeDtypeStruct((NPAD, D), f32),
            jax.ShapeDtypeStruct((NPAD,), f32),
        ],
        mesh=mesh,
        scratch_types=[
            pltpu.VMEM((NC_NODE, CH), jnp.int32),
            pltpu.VMEM((NC_NODE, CH), jnp.int32),
            pltpu.VMEM((CH, D), f32),
            pltpu.VMEM((CH, D), f32),
            pltpu.VMEM((CH, D), f32),
            pltpu.VMEM((CH,), f32),
            pltpu.VMEM((CH,), f32),
            pltpu.VMEM((CH,), f32),
            pltpu.SemaphoreType.DMA,
            pltpu.SemaphoreType.DMA,
        ],
    )(v_p, a_p, hu_tab, att_flat)

    BL3 = 8192
    rel_col = jnp.pad(rel, (0, EPAD - E)).reshape(EPAD, 1)
    eg_col = jnp.pad(eg, (0, EPAD - E)).reshape(EPAD, 1)
    msg = pl.pallas_call(
        _tc_msg_body,
        grid=(EPAD // BL3,),
        in_specs=[
            pl.BlockSpec((BL3, D), lambda i: (i, 0)),
            pl.BlockSpec((BL3, D), lambda i: (i, 0)),
            pl.BlockSpec((BL3, 1), lambda i: (i, 0)),
            pl.BlockSpec((BL3, 1), lambda i: (i, 0)),
            _full((512, D)), _full((B, D)),
            _full((D, D)), _full((1, D)),
        ],
        out_specs=pl.BlockSpec((BL3, D), lambda i: (i, 0)),
        out_shape=jax.ShapeDtypeStruct((EPAD, D), f32),
    )(pre, pre2, rel_col, eg_col, rt, qm1, W_m2, bm2)

    seg, cntp = pl.kernel(
        _sc_scatter_body,
        out_type=[
            jax.ShapeDtypeStruct((2, NSEG, D), f32),
            jax.ShapeDtypeStruct((2, NSEG), f32),
        ],
        mesh=mesh,
        scratch_types=[
            pltpu.VMEM((CH, D), f32),
            pltpu.VMEM((CH, D), f32),
            pltpu.VMEM((EC, CH), jnp.int32),
            pltpu.VMEM((CH,), f32),
            pltpu.VMEM((640,), f32),
            pltpu.VMEM_SHARED((NSEG, D), f32),
            pltpu.VMEM_SHARED((NSEG,), f32),
            pltpu.SemaphoreType.DMA,
            pltpu.SemaphoreType.DMA,
        ],
    )(msg, j_s)

    BL5 = 1024
    nblk = NSEG // BL5
    cnt_flat = cntp.reshape(2 * NSEG, 1)
    out = pl.pallas_call(
        _tc_final_body,
        grid=(nblk,),
        in_specs=[
            pl.BlockSpec((2, BL5, D), lambda i: (0, i, 0)),
            pl.BlockSpec((BL5, 1), lambda i: (i, 0)),
            pl.BlockSpec((BL5, 1), lambda i: (i + nblk, 0)),
            pl.BlockSpec((BL5, D), lambda i: (i, 0)),
            pl.BlockSpec((BL5, D), lambda i: (i, 0)),
            pl.BlockSpec((BL5, 1), lambda i: (i, 0)),
            pl.BlockSpec((BL5, 1), lambda i: (i, 0)),
            _full((D, D)), _full((D, D)), _full((D, D)),
            _full((B, D)), _full((D, D)), _full((1, D)),
        ],
        out_specs=pl.BlockSpec((BL5, D), lambda i: (i, 0)),
        out_shape=jax.ShapeDtypeStruct((NSEG, D), f32),
    )(seg, cnt_flat, cnt_flat, hid_pad, hu_g, att_g.reshape(-1, 1), meg2,
      W_h1[0:D], W_h1[D:2 * D], wic, qh1, W_h2, bh2)

    return out[:N_MEM]
